# Initial kernel scaffold; baseline (speedup 1.0000x reference)
import functools

import jax
import jax.numpy as jnp
from jax import lax
from jax.experimental import pallas as pl
from jax.experimental.pallas import tpu as pltpu

_BM = 1000


def _mm_body(a_ref, w_ref, scale_ref, bias_ref, attn_ref, out_ref, e_ref,
             *, has_scale, has_bias, has_attn):
    a = a_ref[...]
    if has_scale:
        a = a * scale_ref[...][:, None]
    acc = jnp.dot(a, w_ref[...], preferred_element_type=jnp.float32)
    if has_bias:
        acc = acc + bias_ref[...][None, :]
    out_ref[...] = acc
    if has_attn:
        e_ref[...] = jnp.sum(acc * attn_ref[...][None, :], axis=1)


def _mm(a, w, *, scale=None, bias=None, attn=None):
    m, k = a.shape
    d = w.shape[1]
    has_scale = scale is not None
    has_bias = bias is not None
    has_attn = attn is not None
    grid = (m // _BM,)
    in_specs = [
        pl.BlockSpec((_BM, k), lambda i: (i, 0)),
        pl.BlockSpec((k, d), lambda i: (0, 0)),
        pl.BlockSpec((_BM,), lambda i: (i,)) if has_scale else pl.BlockSpec((8,), lambda i: (0,)),
        pl.BlockSpec((d,), lambda i: (0,)) if has_bias else pl.BlockSpec((8,), lambda i: (0,)),
        pl.BlockSpec((d,), lambda i: (0,)) if has_attn else pl.BlockSpec((8,), lambda i: (0,)),
    ]
    out_shapes = [jax.ShapeDtypeStruct((m, d), jnp.float32),
                  jax.ShapeDtypeStruct((m,), jnp.float32)]
    out_specs = [pl.BlockSpec((_BM, d), lambda i: (i, 0)),
                 pl.BlockSpec((_BM,), lambda i: (i,))]
    dummy = jnp.zeros((8,), jnp.float32)
    out, e = pl.pallas_call(
        functools.partial(_mm_body, has_scale=has_scale, has_bias=has_bias,
                          has_attn=has_attn),
        grid=grid,
        in_specs=in_specs,
        out_specs=out_specs,
        out_shape=out_shapes,
    )(a, w,
      scale if has_scale else dummy,
      bias if has_bias else dummy,
      attn if has_attn else dummy)
    if has_attn:
        return out, e
    return out


def _scale_bias_body(z_ref, s_ref, b_ref, o_ref):
    o_ref[...] = z_ref[...] * s_ref[...][:, None] + b_ref[...][None, :]


def _scale_bias(z, s, b):
    m, d = z.shape
    return pl.pallas_call(
        _scale_bias_body,
        grid=(m // _BM,),
        in_specs=[pl.BlockSpec((_BM, d), lambda i: (i, 0)),
                  pl.BlockSpec((_BM,), lambda i: (i,)),
                  pl.BlockSpec((d,), lambda i: (0,))],
        out_specs=pl.BlockSpec((_BM, d), lambda i: (i, 0)),
        out_shape=jax.ShapeDtypeStruct((m, d), jnp.float32),
    )(z, s, b)


def _edge_softmax(e, dst, num_dst):
    m = jax.ops.segment_max(e, dst, num_segments=num_dst)
    m = jnp.where(jnp.isfinite(m), m, 0.0)
    ex = jnp.exp(e - m[dst])
    s = jax.ops.segment_sum(ex, dst, num_segments=num_dst)
    return ex / s[dst]


def kernel(feat_N, feat_Q, edge_nn, edge_np, edge_pq, W_gcn, b_gcn, W_src1,
           W_dst1, attn_l1, attn_r1, bias1, W_src2, W_dst2, attn_l2, attn_r2,
           bias2):
    n_n = feat_N.shape[0]
    n_p = n_n
    n_q = feat_Q.shape[0]

    src, dst = edge_nn[0], edge_nn[1]
    ones = jnp.ones((src.shape[0],), dtype=jnp.float32)
    out_deg = jnp.clip(jax.ops.segment_sum(ones, src, num_segments=n_n), 1.0, None)
    in_deg = jnp.clip(jax.ops.segment_sum(ones, dst, num_segments=n_n), 1.0, None)
    os_ = out_deg ** -0.5
    is_ = in_deg ** -0.5
    y = _mm(feat_N, W_gcn, scale=os_)
    z = jax.ops.segment_sum(y[src], dst, num_segments=n_n)
    h = _scale_bias(z, is_, b_gcn)

    s1, d1 = edge_np[0], edge_np[1]
    hp = jax.ops.segment_max(h[s1], d1, num_segments=n_p)
    hp = jnp.where(jnp.isfinite(hp), hp, 0.0)
    hp = jax.nn.relu(hp)

    fs, el = _mm(h, W_src1, attn=attn_l1)
    _fd, er = _mm(hp, W_dst1, attn=attn_r1)
    e = jax.nn.leaky_relu(el[s1] + er[d1], 0.2)
    a = _edge_softmax(e, d1, n_p)
    hp2 = jax.ops.segment_sum(fs[s1] * a[:, None], d1, num_segments=n_p) + bias1

    s2, d2 = edge_pq[0], edge_pq[1]
    fs2, el2 = _mm(hp2, W_src2, attn=attn_l2)
    _fd2, er2 = _mm(feat_Q, W_dst2, attn=attn_r2)
    e2 = jax.nn.leaky_relu(el2[s2] + er2[d2], 0.2)
    a2 = _edge_softmax(e2, d2, n_q)
    out = jax.ops.segment_sum(fs2[s2] * a2[:, None], d2, num_segments=n_q) + bias2
    return out, a2


# speedup vs baseline: 1.0341x; 1.0341x over previous
import functools

import jax
import jax.numpy as jnp
from jax import lax
from jax.experimental import pallas as pl
from jax.experimental.pallas import tpu as pltpu

_BM = 1024


def _mm_body(a_ref, w_ref, scale_ref, bias_ref, attn_ref, out_ref, e_ref,
             *, has_scale, has_bias, has_attn):
    a = a_ref[...]
    if has_scale:
        a = a * scale_ref[...][:, None]
    acc = jnp.dot(a, w_ref[...], preferred_element_type=jnp.float32)
    if has_bias:
        acc = acc + bias_ref[...][None, :]
    out_ref[...] = acc
    if has_attn:
        e_ref[...] = jnp.sum(acc * attn_ref[...][None, :], axis=1)


def _mm(a, w, *, scale=None, bias=None, attn=None):
    m, k = a.shape
    d = w.shape[1]
    has_scale = scale is not None
    has_bias = bias is not None
    has_attn = attn is not None
    grid = (m // _BM,)
    in_specs = [
        pl.BlockSpec((_BM, k), lambda i: (i, 0)),
        pl.BlockSpec((k, d), lambda i: (0, 0)),
        pl.BlockSpec((_BM,), lambda i: (i,)) if has_scale else pl.BlockSpec((128,), lambda i: (0,)),
        pl.BlockSpec((d,), lambda i: (0,)) if has_bias else pl.BlockSpec((128,), lambda i: (0,)),
        pl.BlockSpec((d,), lambda i: (0,)) if has_attn else pl.BlockSpec((128,), lambda i: (0,)),
    ]
    out_shapes = [jax.ShapeDtypeStruct((m, d), jnp.float32),
                  jax.ShapeDtypeStruct((m,), jnp.float32)]
    out_specs = [pl.BlockSpec((_BM, d), lambda i: (i, 0)),
                 pl.BlockSpec((_BM,), lambda i: (i,))]
    dummy = jnp.zeros((128,), jnp.float32)
    out, e = pl.pallas_call(
        functools.partial(_mm_body, has_scale=has_scale, has_bias=has_bias,
                          has_attn=has_attn),
        grid=grid,
        in_specs=in_specs,
        out_specs=out_specs,
        out_shape=out_shapes,
    )(a, w,
      scale if has_scale else dummy,
      bias if has_bias else dummy,
      attn if has_attn else dummy)
    if has_attn:
        return out, e
    return out


def _scale_bias_body(z_ref, s_ref, b_ref, o_ref):
    o_ref[...] = z_ref[...] * s_ref[...][:, None] + b_ref[...][None, :]


def _scale_bias(z, s, b):
    m, d = z.shape
    return pl.pallas_call(
        _scale_bias_body,
        grid=(m // _BM,),
        in_specs=[pl.BlockSpec((_BM, d), lambda i: (i, 0)),
                  pl.BlockSpec((_BM,), lambda i: (i,)),
                  pl.BlockSpec((d,), lambda i: (0,))],
        out_specs=pl.BlockSpec((_BM, d), lambda i: (i, 0)),
        out_shape=jax.ShapeDtypeStruct((m, d), jnp.float32),
    )(z, s, b)


def _edge_softmax(e, dst, num_dst):
    m = jax.ops.segment_max(e, dst, num_segments=num_dst)
    m = jnp.where(jnp.isfinite(m), m, 0.0)
    ex = jnp.exp(e - m[dst])
    s = jax.ops.segment_sum(ex, dst, num_segments=num_dst)
    return ex / s[dst]


def kernel(feat_N, feat_Q, edge_nn, edge_np, edge_pq, W_gcn, b_gcn, W_src1,
           W_dst1, attn_l1, attn_r1, bias1, W_src2, W_dst2, attn_l2, attn_r2,
           bias2):
    n_n = feat_N.shape[0]
    mp = ((n_n + _BM - 1) // _BM) * _BM
    pad = mp - n_n
    featN_p = jnp.pad(feat_N, ((0, pad), (0, 0)))
    featQ_p = jnp.pad(feat_Q, ((0, pad), (0, 0)))

    src, dst = edge_nn[0], edge_nn[1]
    ones = jnp.ones((src.shape[0],), dtype=jnp.float32)
    out_deg = jnp.clip(jax.ops.segment_sum(ones, src, num_segments=mp), 1.0, None)
    in_deg = jnp.clip(jax.ops.segment_sum(ones, dst, num_segments=mp), 1.0, None)
    os_ = out_deg ** -0.5
    is_ = in_deg ** -0.5
    y = _mm(featN_p, W_gcn, scale=os_)
    z = jax.ops.segment_sum(y[src], dst, num_segments=mp)
    h = _scale_bias(z, is_, b_gcn)

    s1, d1 = edge_np[0], edge_np[1]
    hp = jax.ops.segment_max(h[s1], d1, num_segments=mp)
    hp = jnp.where(jnp.isfinite(hp), hp, 0.0)
    hp = jax.nn.relu(hp)

    fs, el = _mm(h, W_src1, attn=attn_l1)
    _fd, er = _mm(hp, W_dst1, attn=attn_r1)
    e = jax.nn.leaky_relu(el[s1] + er[d1], 0.2)
    a = _edge_softmax(e, d1, mp)
    hp2 = jax.ops.segment_sum(fs[s1] * a[:, None], d1, num_segments=mp) + bias1

    s2, d2 = edge_pq[0], edge_pq[1]
    fs2, el2 = _mm(hp2, W_src2, attn=attn_l2)
    _fd2, er2 = _mm(featQ_p, W_dst2, attn=attn_r2)
    e2 = jax.nn.leaky_relu(el2[s2] + er2[d2], 0.2)
    a2 = _edge_softmax(e2, d2, mp)
    out = jax.ops.segment_sum(fs2[s2] * a2[:, None], d2, num_segments=mp) + bias2
    return out[:n_n], a2


# speedup vs baseline: 2.4311x; 2.3510x over previous
import functools

import jax
import jax.numpy as jnp
from jax import lax
from jax.experimental import pallas as pl
from jax.experimental.pallas import tpu as pltpu
from jax.experimental.pallas import tpu_sc as plsc

_BM = 1024
_NC, _NS, _L = 2, 16, 16
_NW = _NC * _NS

_SC_PARAMS = pltpu.CompilerParams(needs_layout_passes=False)


@functools.cache
def _sc_mesh():
    return plsc.VectorSubcoreMesh(core_axis_name="c", subcore_axis_name="s",
                                  num_cores=_NC, num_subcores=_NS)


def _sc_degree_hists(src, dst, mp):
    e_pw = src.shape[0] // _NW

    @functools.partial(
        pl.kernel,
        out_type=[jax.ShapeDtypeStruct((_NW, mp), jnp.float32),
                  jax.ShapeDtypeStruct((_NW, mp), jnp.float32)],
        mesh=_sc_mesh(),
        compiler_params=_SC_PARAMS,
        scratch_types=[pltpu.VMEM((e_pw,), jnp.int32),
                       pltpu.VMEM((e_pw,), jnp.int32),
                       pltpu.VMEM((mp,), jnp.float32),
                       pltpu.VMEM((mp,), jnp.float32)],
    )
    def deg_kernel(src_hbm, dst_hbm, osrc_hbm, odst_hbm, sbuf, dbuf, hs, hd):
        c = lax.axis_index("c")
        s = lax.axis_index("s")
        w = s * _NC + c

        def zero_body(i, _):
            z = jnp.zeros((_L,), jnp.float32)
            hs[pl.ds(i * _L, _L)] = z
            hd[pl.ds(i * _L, _L)] = z
            return 0

        lax.fori_loop(0, mp // _L, zero_body, 0)
        pltpu.sync_copy(src_hbm.at[pl.ds(w * e_pw, e_pw)], sbuf)
        pltpu.sync_copy(dst_hbm.at[pl.ds(w * e_pw, e_pw)], dbuf)
        ones = jnp.full((_L,), 1.0, jnp.float32)

        def body(i, _):
            sv = sbuf[pl.ds(i * _L, _L)]
            dv = dbuf[pl.ds(i * _L, _L)]
            plsc.addupdate_scatter(hs, [sv], ones)
            plsc.addupdate_scatter(hd, [dv], ones)
            return 0

        lax.fori_loop(0, e_pw // _L, body, 0)
        pltpu.sync_copy(hs, osrc_hbm.at[w])
        pltpu.sync_copy(hd, odst_hbm.at[w])

    return deg_kernel(src, dst)


def _sc_gather_segsum(tab, src2, dst, ex=None, s_sum=None, bias=None,
                      emit_a=False, edge_split=False):
    if edge_split:
        mp, dh = tab.shape
        two_mp = 2 * mp
    else:
        two_mp, dh = tab.shape
        mp = two_mp // 2
    ep = dst.shape[0]
    t_pw = ep // _NW if edge_split else ep // _NS
    nch = t_pw // 128
    rpt = mp // _NS
    weighted = ex is not None
    has_bias = bias is not None

    out_types = [jax.ShapeDtypeStruct((two_mp, dh), jnp.float32)]
    if emit_a:
        out_types.append(jax.ShapeDtypeStruct((ep,), jnp.float32))
    scratch = [pltpu.VMEM((128,), jnp.int32),
               pltpu.VMEM((128,), jnp.int32),
               pltpu.VMEM((128, dh), jnp.float32),
               pltpu.VMEM((128,), jnp.float32),
               pltpu.VMEM((128,), jnp.float32),
               pltpu.VMEM((mp,), jnp.float32),
               pltpu.VMEM((dh,), jnp.float32),
               pltpu.VMEM((128, dh), jnp.float32),
               pltpu.VMEM_SHARED((mp, dh), jnp.float32),
               pltpu.SemaphoreType.DMA]

    def body(*args):
        if weighted:
            if has_bias:
                (tab_h, src2_h, dst_h, ex_h, ssum_h, bias_h), rest = args[:6], args[6:]
            else:
                (tab_h, src2_h, dst_h, ex_h, ssum_h), rest = args[:5], args[5:]
                bias_h = None
        else:
            if has_bias:
                (tab_h, src2_h, dst_h, bias_h), rest = args[:4], args[4:]
            else:
                (tab_h, src2_h, dst_h), rest = args[:3], args[3:]
                bias_h = None
            ex_h = ssum_h = None
        if emit_a:
            out_h, a_h = rest[0], rest[1]
            rest = rest[2:]
        else:
            out_h = rest[0]
            a_h = None
            rest = rest[1:]
        (sidx, didx, rows, exbuf, abuf, sv, biasv, zbuf, acc, sem) = rest

        c = lax.axis_index("c")
        s = lax.axis_index("s")

        zero16 = jnp.zeros((_L,), jnp.float32)
        if has_bias and edge_split:
            pltpu.sync_copy(bias_h, biasv)
            bvals = [jnp.where(c == 0, biasv[pl.ds(j * _L, _L)], zero16)
                     for j in range(dh // _L)]
        elif has_bias:
            pltpu.sync_copy(bias_h.at[pl.ds(c * dh, dh)], biasv)
            bvals = [biasv[pl.ds(j * _L, _L)] for j in range(dh // _L)]
        else:
            bvals = [zero16] * (dh // _L)

        def zrow(i, _):
            for j in range(dh // _L):
                zbuf[i, pl.ds(j * _L, _L)] = bvals[j]
            return 0

        lax.fori_loop(0, 128, zrow, 0)
        for k in range(rpt // 128):
            pltpu.sync_copy(zbuf, acc.at[pl.ds(s * rpt + k * 128, 128)])
        plsc.subcore_barrier()

        if weighted:
            pltpu.sync_copy(ssum_h, sv)

        base = c * (ep // 2) + s * t_pw if edge_split else s * t_pw

        def chunk(g, _):
            off = base + g * 128
            pltpu.sync_copy(src2_h.at[c, pl.ds(off, 128)], sidx)
            pltpu.sync_copy(dst_h.at[pl.ds(off, 128)], didx)
            pltpu.async_copy(tab_h.at[sidx], rows, sem).wait()
            if weighted:
                pltpu.sync_copy(ex_h.at[pl.ds(off, 128)], exbuf)
                for j in range(8):
                    dv = didx[pl.ds(j * _L, _L)]
                    sg = plsc.load_gather(sv, [dv])
                    abuf[pl.ds(j * _L, _L)] = exbuf[pl.ds(j * _L, _L)] / sg
                if emit_a and edge_split:
                    pltpu.sync_copy(abuf, a_h.at[pl.ds(off, 128)])
                elif emit_a:
                    @pl.when(c == 0)
                    def _():
                        pltpu.sync_copy(abuf, a_h.at[pl.ds(off, 128)])

                def escale(e, _):
                    aw = plsc.load_gather(abuf, [jnp.full((_L,), 0, jnp.int32) + e])
                    for j in range(dh // _L):
                        rows[e, pl.ds(j * _L, _L)] = rows[e, pl.ds(j * _L, _L)] * aw
                    return 0

                lax.fori_loop(0, 128, escale, 0)
            pltpu.sync_copy(rows, acc.at[didx], add=True)
            return 0

        lax.fori_loop(0, nch, chunk, 0)
        plsc.subcore_barrier()
        for k in range(rpt // 128):
            r0 = s * rpt + k * 128
            pltpu.sync_copy(acc.at[pl.ds(r0, 128)],
                            out_h.at[pl.ds(c * mp + r0, 128)])

    fn = pl.kernel(body, out_type=out_types, mesh=_sc_mesh(),
                   compiler_params=_SC_PARAMS, scratch_types=scratch)
    ins = [tab, src2, dst]
    if weighted:
        ins += [ex, s_sum]
    if has_bias:
        ins.append(bias)
    res = fn(*ins)
    return res if emit_a else res[0]


def _sc_edge_softmax_num(el, er, src, dst):
    mp = el.shape[0]
    ep = src.shape[0]
    e_pw = ep // _NW

    def body(el_h, er_h, src_h, dst_h, ex_h, hist_h, elv, erv, sbuf, dbuf,
             exbuf, hist):
        c = lax.axis_index("c")
        s = lax.axis_index("s")
        w = s * _NC + c

        def zero_body(i, _):
            hist[pl.ds(i * _L, _L)] = jnp.zeros((_L,), jnp.float32)
            return 0

        lax.fori_loop(0, mp // _L, zero_body, 0)
        pltpu.sync_copy(el_h, elv)
        pltpu.sync_copy(er_h, erv)
        pltpu.sync_copy(src_h.at[pl.ds(w * e_pw, e_pw)], sbuf)
        pltpu.sync_copy(dst_h.at[pl.ds(w * e_pw, e_pw)], dbuf)

        def body_i(i, _):
            sv = sbuf[pl.ds(i * _L, _L)]
            dv = dbuf[pl.ds(i * _L, _L)]
            x = plsc.load_gather(elv, [sv]) + plsc.load_gather(erv, [dv])
            x = jnp.maximum(x, 0.2 * x)
            exv = jnp.exp(x)
            exbuf[pl.ds(i * _L, _L)] = exv
            plsc.addupdate_scatter(hist, [dv], exv)
            return 0

        lax.fori_loop(0, e_pw // _L, body_i, 0)
        pltpu.sync_copy(exbuf, ex_h.at[pl.ds(w * e_pw, e_pw)])
        pltpu.sync_copy(hist, hist_h.at[w])

    fn = pl.kernel(
        body,
        out_type=[jax.ShapeDtypeStruct((ep,), jnp.float32),
                  jax.ShapeDtypeStruct((_NW, mp), jnp.float32)],
        mesh=_sc_mesh(),
        compiler_params=_SC_PARAMS,
        scratch_types=[pltpu.VMEM((mp,), jnp.float32),
                       pltpu.VMEM((mp,), jnp.float32),
                       pltpu.VMEM((e_pw,), jnp.int32),
                       pltpu.VMEM((e_pw,), jnp.int32),
                       pltpu.VMEM((e_pw,), jnp.float32),
                       pltpu.VMEM((mp,), jnp.float32)])
    return fn(el, er, src, dst)


def _sc_segmax_relu(tab, src, dst):
    mp, d = tab.shape
    ep = src.shape[0]
    rpw = mp // _NW
    ce = 2048
    nst = ep // ce
    ndj = d // _L

    def body(tab_h, src_h, dst_h, out_h, sbuf, dbuf, msrc, mdl, grows, acc,
             sem):
        c = lax.axis_index("c")
        s = lax.axis_index("s")
        w = s * _NC + c
        lo = w * rpw

        def zrow(i, _):
            for j in range(ndj):
                acc[i, pl.ds(j * _L, _L)] = jnp.zeros((_L,), jnp.float32)
            return 0

        lax.fori_loop(0, rpw, zrow, 0)

        def zidx(i, _):
            msrc[pl.ds(i * _L, _L)] = jnp.zeros((_L,), jnp.int32)
            return 0

        lax.fori_loop(0, ce // _L, zidx, 0)

        cols = [lax.iota(jnp.int32, _L) + j * _L for j in range(ndj)]

        def stage(st, _):
            soff = st * ce
            pltpu.sync_copy(src_h.at[pl.ds(soff, ce)], sbuf)
            pltpu.sync_copy(dst_h.at[pl.ds(soff, ce)], dbuf)

            def filt(j, cnt):
                sv = sbuf[pl.ds(j * _L, _L)]
                dl = dbuf[pl.ds(j * _L, _L)] - lo
                mask = (dl >= 0) & (dl < rpw)
                pos = cnt + plsc.cumsum(jnp.where(mask, 1, 0)) - 1
                plsc.store_scatter(msrc, [pos], sv, mask=mask)
                plsc.store_scatter(mdl, [pos], dl, mask=mask)
                return cnt + plsc.all_reduce_population_count(mask)

            cntv = lax.fori_loop(0, ce // _L, filt,
                                 jnp.zeros((_L,), jnp.int32))
            cnt = jnp.max(cntv)

            def group(gi, _):
                gb = gi * 64
                pltpu.async_copy(tab_h.at[msrc.at[pl.ds(gb, 64)]], grows,
                                 sem).wait()
                nloc = jnp.minimum(64, cnt - gb)

                def upd(e, _):
                    dlw = plsc.load_gather(mdl, [jnp.full((_L,), 0, jnp.int32)
                                                 + (gb + e)])
                    for j in range(ndj):
                        cur = plsc.load_gather(acc, [dlw, cols[j]])
                        valv = grows[e, pl.ds(j * _L, _L)]
                        plsc.store_scatter(acc, [dlw, cols[j]],
                                           jnp.maximum(cur, valv))
                    return 0

                lax.fori_loop(0, nloc, upd, 0)
                return 0

            lax.fori_loop(0, (cnt + 63) // 64, group, 0)
            return 0

        lax.fori_loop(0, nst, stage, 0)
        pltpu.sync_copy(acc, out_h.at[pl.ds(lo, rpw)])

    fn = pl.kernel(
        body,
        out_type=jax.ShapeDtypeStruct((mp, d), jnp.float32),
        mesh=_sc_mesh(),
        compiler_params=_SC_PARAMS,
        scratch_types=[pltpu.VMEM((ce,), jnp.int32),
                       pltpu.VMEM((ce,), jnp.int32),
                       pltpu.VMEM((ce,), jnp.int32),
                       pltpu.VMEM((ce,), jnp.int32),
                       pltpu.VMEM((64, d), jnp.float32),
                       pltpu.VMEM((rpw, d), jnp.float32),
                       pltpu.SemaphoreType.DMA])
    return fn(tab, src, dst)


def _add2_body(a_ref, b_ref, o_ref):
    o_ref[...] = a_ref[...] + b_ref[...]


def _add2(a, b):
    m, d = a.shape
    return pl.pallas_call(
        _add2_body,
        grid=(m // _BM,),
        in_specs=[pl.BlockSpec((_BM, d), lambda i: (i, 0)),
                  pl.BlockSpec((_BM, d), lambda i: (i, 0))],
        out_specs=pl.BlockSpec((_BM, d), lambda i: (i, 0)),
        out_shape=jax.ShapeDtypeStruct((m, d), jnp.float32),
    )(a, b)


def _colsum_body(h_ref, o_ref):
    o_ref[...] = jnp.sum(h_ref[...], axis=0)


def _colsum(h):
    nw, m = h.shape
    return pl.pallas_call(
        _colsum_body,
        grid=(m // _BM,),
        in_specs=[pl.BlockSpec((nw, _BM), lambda i: (0, i))],
        out_specs=pl.BlockSpec((_BM,), lambda i: (i,)),
        out_shape=jax.ShapeDtypeStruct((m,), jnp.float32),
    )(h)


def _mm_body(a_ref, w_ref, scale_ref, bias_ref, attn_ref, out_ref, e_ref,
             *, has_scale, has_bias, has_attn):
    a = a_ref[...]
    if has_scale:
        s = lax.rsqrt(jnp.clip(jnp.sum(scale_ref[...], axis=0), 1.0, None))
        a = a * s[:, None]
    acc = jnp.dot(a, w_ref[...], preferred_element_type=jnp.float32)
    if has_bias:
        acc = acc + bias_ref[...][None, :]
    out_ref[...] = acc
    if has_attn:
        e_ref[...] = jnp.sum(acc * attn_ref[...][None, :], axis=1)


def _mm(a, w, *, scale=None, bias=None, attn=None):
    m, k = a.shape
    d = w.shape[1]
    has_scale = scale is not None
    has_bias = bias is not None
    has_attn = attn is not None
    grid = (m // _BM,)
    in_specs = [
        pl.BlockSpec((_BM, k), lambda i: (i, 0)),
        pl.BlockSpec((k, d), lambda i: (0, 0)),
        pl.BlockSpec((_NW, _BM), lambda i: (0, i)) if has_scale else pl.BlockSpec((128,), lambda i: (0,)),
        pl.BlockSpec((d,), lambda i: (0,)) if has_bias else pl.BlockSpec((128,), lambda i: (0,)),
        pl.BlockSpec((d,), lambda i: (0,)) if has_attn else pl.BlockSpec((128,), lambda i: (0,)),
    ]
    out_shapes = [jax.ShapeDtypeStruct((m, d), jnp.float32),
                  jax.ShapeDtypeStruct((m,), jnp.float32)]
    out_specs = [pl.BlockSpec((_BM, d), lambda i: (i, 0)),
                 pl.BlockSpec((_BM,), lambda i: (i,))]
    dummy = jnp.zeros((128,), jnp.float32)
    out, e = pl.pallas_call(
        functools.partial(_mm_body, has_scale=has_scale, has_bias=has_bias,
                          has_attn=has_attn),
        grid=grid,
        in_specs=in_specs,
        out_specs=out_specs,
        out_shape=out_shapes,
    )(a, w,
      scale if has_scale else dummy,
      bias if has_bias else dummy,
      attn if has_attn else dummy)
    if has_attn:
        return out, e
    return out


def _scale_bias_body(z_ref, s_ref, b_ref, o_ref):
    s = lax.rsqrt(jnp.clip(jnp.sum(s_ref[...], axis=0), 1.0, None))
    o_ref[...] = z_ref[...] * s[:, None] + b_ref[...][None, :]


def _scale_bias(z, s, b):
    m, d = z.shape
    return pl.pallas_call(
        _scale_bias_body,
        grid=(m // _BM,),
        in_specs=[pl.BlockSpec((_BM, d), lambda i: (i, 0)),
                  pl.BlockSpec((_NW, _BM), lambda i: (0, i)),
                  pl.BlockSpec((d,), lambda i: (0,))],
        out_specs=pl.BlockSpec((_BM, d), lambda i: (i, 0)),
        out_shape=jax.ShapeDtypeStruct((m, d), jnp.float32),
    )(z, s, b)


def _edge_softmax(e, dst, num_dst):
    m = jax.ops.segment_max(e, dst, num_segments=num_dst)
    m = jnp.where(jnp.isfinite(m), m, 0.0)
    ex = jnp.exp(e - m[dst])
    s = jax.ops.segment_sum(ex, dst, num_segments=num_dst)
    return ex / s[dst]


def kernel(feat_N, feat_Q, edge_nn, edge_np, edge_pq, W_gcn, b_gcn, W_src1,
           W_dst1, attn_l1, attn_r1, bias1, W_src2, W_dst2, attn_l2, attn_r2,
           bias2):
    n_n = feat_N.shape[0]
    mp = ((n_n + _BM - 1) // _BM) * _BM
    pad = mp - n_n
    featN_p = jnp.pad(feat_N, ((0, pad), (0, 0)))
    featQ_p = jnp.pad(feat_Q, ((0, pad), (0, 0)))

    def pad_edges(e2):
        s, d = e2[0], e2[1]
        epad = (-s.shape[0]) % (_NW * 128)
        sp = jnp.pad(s, (0, epad), constant_values=n_n)
        dp = jnp.pad(d, (0, epad), constant_values=mp - 1)
        return sp, dp, jnp.stack([sp, sp + mp])

    def split2(x):
        dh = x.shape[1] // 2
        return jnp.concatenate([x[:, :dh], x[:, dh:]], axis=0)

    def join2(x):
        mp_ = x.shape[0] // 2
        return jnp.concatenate([x[:mp_], x[mp_:]], axis=1)

    src_p, dst_p, src2 = pad_edges(edge_nn)
    hs_, hd_ = _sc_degree_hists(src_p, dst_p, mp)
    y = _mm(featN_p, W_gcn, scale=hs_)
    z = join2(_sc_gather_segsum(split2(y), src2, dst_p))
    h = _scale_bias(z, hd_, b_gcn)

    s1p, d1p, s1_2 = pad_edges(edge_np)
    hp = _sc_segmax_relu(h, s1p, d1p)

    fs, el = _mm(h, W_src1, attn=attn_l1)
    _fd, er = _mm(hp, W_dst1, attn=attn_r1)
    ex1, sh1 = _sc_edge_softmax_num(el, er, s1p, d1p)
    hp2 = join2(_sc_gather_segsum(split2(fs), s1_2, d1p, ex=ex1,
                                  s_sum=_colsum(sh1), bias=bias1))

    s2p, d2p, s2_2 = pad_edges(edge_pq)
    fs2, el2 = _mm(hp2, W_src2, attn=attn_l2)
    _fd2, er2 = _mm(featQ_p, W_dst2, attn=attn_r2)
    ex2, sh2 = _sc_edge_softmax_num(el2, er2, s2p, d2p)
    outf, a2p = _sc_gather_segsum(fs2, jnp.stack([s2p, s2p]), d2p, ex=ex2,
                                  s_sum=_colsum(sh2), bias=bias2, emit_a=True,
                                  edge_split=True)
    out = _add2(outf[:mp], outf[mp:])
    return out[:n_n], a2p[:edge_pq.shape[1]]


# speedup vs baseline: 2.4831x; 1.0214x over previous
import functools

import jax
import jax.numpy as jnp
from jax import lax
from jax.experimental import pallas as pl
from jax.experimental.pallas import tpu as pltpu
from jax.experimental.pallas import tpu_sc as plsc

_BM = 1024
_NC, _NS, _L = 2, 16, 16
_NW = _NC * _NS

_SC_PARAMS = pltpu.CompilerParams(needs_layout_passes=False)


@functools.cache
def _sc_mesh():
    return plsc.VectorSubcoreMesh(core_axis_name="c", subcore_axis_name="s",
                                  num_cores=_NC, num_subcores=_NS)


def _sc_degree_hists(src, dst, mp):
    e_pw = src.shape[0] // _NW

    @functools.partial(
        pl.kernel,
        out_type=[jax.ShapeDtypeStruct((_NW, mp), jnp.float32),
                  jax.ShapeDtypeStruct((_NW, mp), jnp.float32)],
        mesh=_sc_mesh(),
        compiler_params=_SC_PARAMS,
        scratch_types=[pltpu.VMEM((e_pw,), jnp.int32),
                       pltpu.VMEM((e_pw,), jnp.int32),
                       pltpu.VMEM((mp,), jnp.float32),
                       pltpu.VMEM((mp,), jnp.float32)],
    )
    def deg_kernel(src_hbm, dst_hbm, osrc_hbm, odst_hbm, sbuf, dbuf, hs, hd):
        c = lax.axis_index("c")
        s = lax.axis_index("s")
        w = s * _NC + c

        def zero_body(i, _):
            z = jnp.zeros((_L,), jnp.float32)
            hs[pl.ds(i * _L, _L)] = z
            hd[pl.ds(i * _L, _L)] = z
            return 0

        lax.fori_loop(0, mp // _L, zero_body, 0)
        pltpu.sync_copy(src_hbm.at[pl.ds(w * e_pw, e_pw)], sbuf)
        pltpu.sync_copy(dst_hbm.at[pl.ds(w * e_pw, e_pw)], dbuf)
        ones = jnp.full((_L,), 1.0, jnp.float32)

        def body(i, _):
            sv = sbuf[pl.ds(i * _L, _L)]
            dv = dbuf[pl.ds(i * _L, _L)]
            plsc.addupdate_scatter(hs, [sv], ones)
            plsc.addupdate_scatter(hd, [dv], ones)
            return 0

        lax.fori_loop(0, e_pw // _L, body, 0)
        pltpu.sync_copy(hs, osrc_hbm.at[w])
        pltpu.sync_copy(hd, odst_hbm.at[w])

    return deg_kernel(src, dst)


def _sc_gather_segsum(tab, src2, dst, ex=None, s_sum=None, bias=None,
                      emit_a=False, edge_split=False):
    if edge_split:
        mp, dh = tab.shape
        two_mp = 2 * mp
    else:
        two_mp, dh = tab.shape
        mp = two_mp // 2
    ep = dst.shape[0]
    t_pw = ep // _NW if edge_split else ep // _NS
    nch = t_pw // 128
    rpt = mp // _NS
    weighted = ex is not None
    has_bias = bias is not None

    out_types = [jax.ShapeDtypeStruct((two_mp, dh), jnp.float32)]
    if emit_a:
        out_types.append(jax.ShapeDtypeStruct((ep,), jnp.float32))
    scratch = [pltpu.VMEM((128,), jnp.int32),
               pltpu.VMEM((128,), jnp.int32),
               pltpu.VMEM((128, dh), jnp.float32),
               pltpu.VMEM((128,), jnp.float32),
               pltpu.VMEM((128,), jnp.float32),
               pltpu.VMEM((mp,), jnp.float32),
               pltpu.VMEM((dh,), jnp.float32),
               pltpu.VMEM((128, dh), jnp.float32),
               pltpu.VMEM_SHARED((mp, dh), jnp.float32),
               pltpu.SemaphoreType.DMA]

    def body(*args):
        if weighted:
            if has_bias:
                (tab_h, src2_h, dst_h, ex_h, ssum_h, bias_h), rest = args[:6], args[6:]
            else:
                (tab_h, src2_h, dst_h, ex_h, ssum_h), rest = args[:5], args[5:]
                bias_h = None
        else:
            if has_bias:
                (tab_h, src2_h, dst_h, bias_h), rest = args[:4], args[4:]
            else:
                (tab_h, src2_h, dst_h), rest = args[:3], args[3:]
                bias_h = None
            ex_h = ssum_h = None
        if emit_a:
            out_h, a_h = rest[0], rest[1]
            rest = rest[2:]
        else:
            out_h = rest[0]
            a_h = None
            rest = rest[1:]
        (sidx, didx, rows, exbuf, abuf, sv, biasv, zbuf, acc, sem) = rest

        c = lax.axis_index("c")
        s = lax.axis_index("s")

        zero16 = jnp.zeros((_L,), jnp.float32)
        if has_bias and edge_split:
            pltpu.sync_copy(bias_h, biasv)
            bvals = [jnp.where(c == 0, biasv[pl.ds(j * _L, _L)], zero16)
                     for j in range(dh // _L)]
        elif has_bias:
            pltpu.sync_copy(bias_h.at[pl.ds(c * dh, dh)], biasv)
            bvals = [biasv[pl.ds(j * _L, _L)] for j in range(dh // _L)]
        else:
            bvals = [zero16] * (dh // _L)

        def zrow(i, _):
            for j in range(dh // _L):
                zbuf[i, pl.ds(j * _L, _L)] = bvals[j]
            return 0

        lax.fori_loop(0, 128, zrow, 0)
        for k in range(rpt // 128):
            pltpu.sync_copy(zbuf, acc.at[pl.ds(s * rpt + k * 128, 128)])
        plsc.subcore_barrier()

        if weighted:
            pltpu.sync_copy(ssum_h, sv)

        base = c * (ep // 2) + s * t_pw if edge_split else s * t_pw

        def chunk(g, _):
            off = base + g * 128
            pltpu.sync_copy(src2_h.at[c, pl.ds(off, 128)], sidx)
            pltpu.sync_copy(dst_h.at[pl.ds(off, 128)], didx)
            pltpu.async_copy(tab_h.at[sidx], rows, sem).wait()
            if weighted:
                pltpu.sync_copy(ex_h.at[pl.ds(off, 128)], exbuf)
                for j in range(8):
                    dv = didx[pl.ds(j * _L, _L)]
                    sg = plsc.load_gather(sv, [dv])
                    abuf[pl.ds(j * _L, _L)] = exbuf[pl.ds(j * _L, _L)] / sg
                if emit_a and edge_split:
                    pltpu.sync_copy(abuf, a_h.at[pl.ds(off, 128)])
                elif emit_a:
                    @pl.when(c == 0)
                    def _():
                        pltpu.sync_copy(abuf, a_h.at[pl.ds(off, 128)])

                def escale(e, _):
                    aw = plsc.load_gather(abuf, [jnp.full((_L,), 0, jnp.int32) + e])
                    scaled = [rows[e, pl.ds(j * _L, _L)] * aw
                              for j in range(dh // _L)]
                    for j in range(dh // _L):
                        rows[e, pl.ds(j * _L, _L)] = scaled[j]
                    return 0

                lax.fori_loop(0, 128, escale, 0)
            pltpu.sync_copy(rows, acc.at[didx], add=True)
            return 0

        lax.fori_loop(0, nch, chunk, 0)
        plsc.subcore_barrier()
        for k in range(rpt // 128):
            r0 = s * rpt + k * 128
            pltpu.sync_copy(acc.at[pl.ds(r0, 128)],
                            out_h.at[pl.ds(c * mp + r0, 128)])

    fn = pl.kernel(body, out_type=out_types, mesh=_sc_mesh(),
                   compiler_params=_SC_PARAMS, scratch_types=scratch)
    ins = [tab, src2, dst]
    if weighted:
        ins += [ex, s_sum]
    if has_bias:
        ins.append(bias)
    res = fn(*ins)
    return res if emit_a else res[0]


def _sc_edge_softmax_num(el, er, src, dst):
    mp = el.shape[0]
    ep = src.shape[0]
    e_pw = ep // _NW

    def body(el_h, er_h, src_h, dst_h, ex_h, hist_h, elv, erv, sbuf, dbuf,
             exbuf, hist):
        c = lax.axis_index("c")
        s = lax.axis_index("s")
        w = s * _NC + c

        def zero_body(i, _):
            hist[pl.ds(i * _L, _L)] = jnp.zeros((_L,), jnp.float32)
            return 0

        lax.fori_loop(0, mp // _L, zero_body, 0)
        pltpu.sync_copy(el_h, elv)
        pltpu.sync_copy(er_h, erv)
        pltpu.sync_copy(src_h.at[pl.ds(w * e_pw, e_pw)], sbuf)
        pltpu.sync_copy(dst_h.at[pl.ds(w * e_pw, e_pw)], dbuf)

        def body_i(i, _):
            sv = sbuf[pl.ds(i * _L, _L)]
            dv = dbuf[pl.ds(i * _L, _L)]
            x = plsc.load_gather(elv, [sv]) + plsc.load_gather(erv, [dv])
            x = jnp.maximum(x, 0.2 * x)
            exv = jnp.exp(x)
            exbuf[pl.ds(i * _L, _L)] = exv
            plsc.addupdate_scatter(hist, [dv], exv)
            return 0

        lax.fori_loop(0, e_pw // _L, body_i, 0)
        pltpu.sync_copy(exbuf, ex_h.at[pl.ds(w * e_pw, e_pw)])
        pltpu.sync_copy(hist, hist_h.at[w])

    fn = pl.kernel(
        body,
        out_type=[jax.ShapeDtypeStruct((ep,), jnp.float32),
                  jax.ShapeDtypeStruct((_NW, mp), jnp.float32)],
        mesh=_sc_mesh(),
        compiler_params=_SC_PARAMS,
        scratch_types=[pltpu.VMEM((mp,), jnp.float32),
                       pltpu.VMEM((mp,), jnp.float32),
                       pltpu.VMEM((e_pw,), jnp.int32),
                       pltpu.VMEM((e_pw,), jnp.int32),
                       pltpu.VMEM((e_pw,), jnp.float32),
                       pltpu.VMEM((mp,), jnp.float32)])
    return fn(el, er, src, dst)


def _sc_segmax_relu(tab, src, dst):
    mp, d = tab.shape
    ep = src.shape[0]
    rpw = mp // _NW
    ce = 2048
    nst = ep // ce
    ndj = d // _L

    def body(tab_h, src_h, dst_h, out_h, sbuf, dbuf, msrc, mdl, grows, acc,
             sem):
        c = lax.axis_index("c")
        s = lax.axis_index("s")
        w = s * _NC + c
        lo = w * rpw

        def zrow(i, _):
            for j in range(ndj):
                acc[i, pl.ds(j * _L, _L)] = jnp.zeros((_L,), jnp.float32)
            return 0

        lax.fori_loop(0, rpw, zrow, 0)

        def zidx(i, _):
            msrc[pl.ds(i * _L, _L)] = jnp.zeros((_L,), jnp.int32)
            mdl[pl.ds(i * _L, _L)] = jnp.zeros((_L,), jnp.int32)
            return 0

        lax.fori_loop(0, ce // _L, zidx, 0)

        cols = [lax.iota(jnp.int32, _L) + j * _L for j in range(ndj)]

        def stage(st, _):
            soff = st * ce
            pltpu.sync_copy(src_h.at[pl.ds(soff, ce)], sbuf)
            pltpu.sync_copy(dst_h.at[pl.ds(soff, ce)], dbuf)

            def filt(j, cnt):
                sv = sbuf[pl.ds(j * _L, _L)]
                dl = dbuf[pl.ds(j * _L, _L)] - lo
                mask = (dl >= 0) & (dl < rpw)
                pos = cnt + plsc.cumsum(jnp.where(mask, 1, 0)) - 1
                plsc.store_scatter(msrc, [pos], sv, mask=mask)
                plsc.store_scatter(mdl, [pos], dl, mask=mask)
                return cnt + plsc.all_reduce_population_count(mask)

            cntv = lax.fori_loop(0, ce // _L, filt,
                                 jnp.zeros((_L,), jnp.int32))
            cnt = jnp.max(cntv)

            def group(gi, _):
                gb = gi * 64
                pltpu.async_copy(tab_h.at[msrc.at[pl.ds(gb, 64)]], grows,
                                 sem).wait()

                nloc = jnp.minimum(64, cnt - gb)

                def upd(e, _):
                    dlw = plsc.load_gather(
                        mdl, [jnp.full((_L,), 0, jnp.int32) + (gb + e)])
                    news = [jnp.maximum(plsc.load_gather(acc, [dlw, cols[j]]),
                                        grows[e, pl.ds(j * _L, _L)])
                            for j in range(ndj)]
                    for j in range(ndj):
                        plsc.store_scatter(acc, [dlw, cols[j]], news[j])
                    return 0

                lax.fori_loop(0, nloc, upd, 0)
                return 0

            lax.fori_loop(0, (cnt + 63) // 64, group, 0)
            return 0

        lax.fori_loop(0, nst, stage, 0)
        pltpu.sync_copy(acc, out_h.at[pl.ds(lo, rpw)])

    fn = pl.kernel(
        body,
        out_type=jax.ShapeDtypeStruct((mp, d), jnp.float32),
        mesh=_sc_mesh(),
        compiler_params=_SC_PARAMS,
        scratch_types=[pltpu.VMEM((ce,), jnp.int32),
                       pltpu.VMEM((ce,), jnp.int32),
                       pltpu.VMEM((ce,), jnp.int32),
                       pltpu.VMEM((ce,), jnp.int32),
                       pltpu.VMEM((64, d), jnp.float32),
                       pltpu.VMEM((rpw, d), jnp.float32),
                       pltpu.SemaphoreType.DMA])
    return fn(tab, src, dst)


def _add2_body(a_ref, b_ref, o_ref):
    o_ref[...] = a_ref[...] + b_ref[...]


def _add2(a, b):
    m, d = a.shape
    return pl.pallas_call(
        _add2_body,
        grid=(m // _BM,),
        in_specs=[pl.BlockSpec((_BM, d), lambda i: (i, 0)),
                  pl.BlockSpec((_BM, d), lambda i: (i, 0))],
        out_specs=pl.BlockSpec((_BM, d), lambda i: (i, 0)),
        out_shape=jax.ShapeDtypeStruct((m, d), jnp.float32),
    )(a, b)


def _colsum_body(h_ref, o_ref):
    o_ref[...] = jnp.sum(h_ref[...], axis=0)


def _colsum(h):
    nw, m = h.shape
    return pl.pallas_call(
        _colsum_body,
        grid=(m // _BM,),
        in_specs=[pl.BlockSpec((nw, _BM), lambda i: (0, i))],
        out_specs=pl.BlockSpec((_BM,), lambda i: (i,)),
        out_shape=jax.ShapeDtypeStruct((m,), jnp.float32),
    )(h)


def _mm_body(a_ref, w_ref, scale_ref, bias_ref, attn_ref, out_ref, e_ref,
             *, has_scale, has_bias, has_attn):
    a = a_ref[...]
    if has_scale:
        s = lax.rsqrt(jnp.clip(jnp.sum(scale_ref[...], axis=0), 1.0, None))
        a = a * s[:, None]
    acc = jnp.dot(a, w_ref[...], preferred_element_type=jnp.float32)
    if has_bias:
        acc = acc + bias_ref[...][None, :]
    out_ref[...] = acc
    if has_attn:
        e_ref[...] = jnp.sum(acc * attn_ref[...][None, :], axis=1)


def _mm(a, w, *, scale=None, bias=None, attn=None):
    m, k = a.shape
    d = w.shape[1]
    has_scale = scale is not None
    has_bias = bias is not None
    has_attn = attn is not None
    grid = (m // _BM,)
    in_specs = [
        pl.BlockSpec((_BM, k), lambda i: (i, 0)),
        pl.BlockSpec((k, d), lambda i: (0, 0)),
        pl.BlockSpec((_NW, _BM), lambda i: (0, i)) if has_scale else pl.BlockSpec((128,), lambda i: (0,)),
        pl.BlockSpec((d,), lambda i: (0,)) if has_bias else pl.BlockSpec((128,), lambda i: (0,)),
        pl.BlockSpec((d,), lambda i: (0,)) if has_attn else pl.BlockSpec((128,), lambda i: (0,)),
    ]
    out_shapes = [jax.ShapeDtypeStruct((m, d), jnp.float32),
                  jax.ShapeDtypeStruct((m,), jnp.float32)]
    out_specs = [pl.BlockSpec((_BM, d), lambda i: (i, 0)),
                 pl.BlockSpec((_BM,), lambda i: (i,))]
    dummy = jnp.zeros((128,), jnp.float32)
    out, e = pl.pallas_call(
        functools.partial(_mm_body, has_scale=has_scale, has_bias=has_bias,
                          has_attn=has_attn),
        grid=grid,
        in_specs=in_specs,
        out_specs=out_specs,
        out_shape=out_shapes,
    )(a, w,
      scale if has_scale else dummy,
      bias if has_bias else dummy,
      attn if has_attn else dummy)
    if has_attn:
        return out, e
    return out


def _scale_bias_body(z_ref, s_ref, b_ref, o_ref):
    s = lax.rsqrt(jnp.clip(jnp.sum(s_ref[...], axis=0), 1.0, None))
    o_ref[...] = z_ref[...] * s[:, None] + b_ref[...][None, :]


def _scale_bias(z, s, b):
    m, d = z.shape
    return pl.pallas_call(
        _scale_bias_body,
        grid=(m // _BM,),
        in_specs=[pl.BlockSpec((_BM, d), lambda i: (i, 0)),
                  pl.BlockSpec((_NW, _BM), lambda i: (0, i)),
                  pl.BlockSpec((d,), lambda i: (0,))],
        out_specs=pl.BlockSpec((_BM, d), lambda i: (i, 0)),
        out_shape=jax.ShapeDtypeStruct((m, d), jnp.float32),
    )(z, s, b)


def _edge_softmax(e, dst, num_dst):
    m = jax.ops.segment_max(e, dst, num_segments=num_dst)
    m = jnp.where(jnp.isfinite(m), m, 0.0)
    ex = jnp.exp(e - m[dst])
    s = jax.ops.segment_sum(ex, dst, num_segments=num_dst)
    return ex / s[dst]


def kernel(feat_N, feat_Q, edge_nn, edge_np, edge_pq, W_gcn, b_gcn, W_src1,
           W_dst1, attn_l1, attn_r1, bias1, W_src2, W_dst2, attn_l2, attn_r2,
           bias2):
    n_n = feat_N.shape[0]
    mp = ((n_n + _BM - 1) // _BM) * _BM
    pad = mp - n_n
    featN_p = jnp.pad(feat_N, ((0, pad), (0, 0)))
    featQ_p = jnp.pad(feat_Q, ((0, pad), (0, 0)))

    def pad_edges(e2):
        s, d = e2[0], e2[1]
        epad = (-s.shape[0]) % (_NW * 128)
        sp = jnp.pad(s, (0, epad), constant_values=n_n)
        dp = jnp.pad(d, (0, epad), constant_values=mp - 1)
        return sp, dp, jnp.stack([sp, sp + mp])

    def split2(x):
        dh = x.shape[1] // 2
        return jnp.concatenate([x[:, :dh], x[:, dh:]], axis=0)

    def join2(x):
        mp_ = x.shape[0] // 2
        return jnp.concatenate([x[:mp_], x[mp_:]], axis=1)

    src_p, dst_p, src2 = pad_edges(edge_nn)
    hs_, hd_ = _sc_degree_hists(src_p, dst_p, mp)
    y = _mm(featN_p, W_gcn, scale=hs_)
    z = join2(_sc_gather_segsum(split2(y), src2, dst_p))
    h = _scale_bias(z, hd_, b_gcn)

    s1p, d1p, s1_2 = pad_edges(edge_np)
    hp = _sc_segmax_relu(h, s1p, d1p)

    fs, el = _mm(h, W_src1, attn=attn_l1)
    _fd, er = _mm(hp, W_dst1, attn=attn_r1)
    ex1, sh1 = _sc_edge_softmax_num(el, er, s1p, d1p)
    hp2 = join2(_sc_gather_segsum(split2(fs), s1_2, d1p, ex=ex1,
                                  s_sum=_colsum(sh1), bias=bias1))

    s2p, d2p, s2_2 = pad_edges(edge_pq)
    fs2, el2 = _mm(hp2, W_src2, attn=attn_l2)
    _fd2, er2 = _mm(featQ_p, W_dst2, attn=attn_r2)
    ex2, sh2 = _sc_edge_softmax_num(el2, er2, s2p, d2p)
    outf, a2p = _sc_gather_segsum(fs2, jnp.stack([s2p, s2p]), d2p, ex=ex2,
                                  s_sum=_colsum(sh2), bias=bias2, emit_a=True,
                                  edge_split=True)
    out = _add2(outf[:mp], outf[mp:])
    return out[:n_n], a2p[:edge_pq.shape[1]]


# speedup vs baseline: 2.6796x; 1.0792x over previous
import functools

import jax
import jax.numpy as jnp
from jax import lax
from jax.experimental import pallas as pl
from jax.experimental.pallas import tpu as pltpu
from jax.experimental.pallas import tpu_sc as plsc

_BM = 1024
_NC, _NS, _L = 2, 16, 16
_NW = _NC * _NS

_SC_PARAMS = pltpu.CompilerParams(needs_layout_passes=False)


@functools.cache
def _sc_mesh():
    return plsc.VectorSubcoreMesh(core_axis_name="c", subcore_axis_name="s",
                                  num_cores=_NC, num_subcores=_NS)


def _sc_degree_hists(src, dst, mp):
    e_pw = src.shape[0] // _NW

    @functools.partial(
        pl.kernel,
        out_type=[jax.ShapeDtypeStruct((_NW, mp), jnp.float32),
                  jax.ShapeDtypeStruct((_NW, mp), jnp.float32)],
        mesh=_sc_mesh(),
        compiler_params=_SC_PARAMS,
        scratch_types=[pltpu.VMEM((e_pw,), jnp.int32),
                       pltpu.VMEM((e_pw,), jnp.int32),
                       pltpu.VMEM((mp,), jnp.float32),
                       pltpu.VMEM((mp,), jnp.float32)],
    )
    def deg_kernel(src_hbm, dst_hbm, osrc_hbm, odst_hbm, sbuf, dbuf, hs, hd):
        c = lax.axis_index("c")
        s = lax.axis_index("s")
        w = s * _NC + c

        def zero_body(i, _):
            z = jnp.zeros((_L,), jnp.float32)
            hs[pl.ds(i * _L, _L)] = z
            hd[pl.ds(i * _L, _L)] = z
            return 0

        lax.fori_loop(0, mp // _L, zero_body, 0)
        pltpu.sync_copy(src_hbm.at[pl.ds(w * e_pw, e_pw)], sbuf)
        pltpu.sync_copy(dst_hbm.at[pl.ds(w * e_pw, e_pw)], dbuf)
        ones = jnp.full((_L,), 1.0, jnp.float32)

        def body(i, _):
            sv = sbuf[pl.ds(i * _L, _L)]
            dv = dbuf[pl.ds(i * _L, _L)]
            plsc.addupdate_scatter(hs, [sv], ones)
            plsc.addupdate_scatter(hd, [dv], ones)
            return 0

        lax.fori_loop(0, e_pw // _L, body, 0)
        pltpu.sync_copy(hs, osrc_hbm.at[w])
        pltpu.sync_copy(hd, odst_hbm.at[w])

    return deg_kernel(src, dst)


def _sc_gather_segsum(tab, src2, dst, ex=None, s_sum=None, bias=None,
                      emit_a=False, edge_split=False):
    if edge_split:
        mp, dh = tab.shape
        two_mp = 2 * mp
    else:
        two_mp, dh = tab.shape
        mp = two_mp // 2
    ep = dst.shape[0]
    t_pw = ep // _NW if edge_split else ep // _NS
    nch = t_pw // 128
    rpt = mp // _NS
    weighted = ex is not None
    has_bias = bias is not None

    ch = 64
    nch = t_pw // ch
    out_types = [jax.ShapeDtypeStruct((two_mp, dh), jnp.float32)]
    if emit_a:
        out_types.append(jax.ShapeDtypeStruct((ep,), jnp.float32))
    scratch = [pltpu.VMEM((ch,), jnp.int32),
               pltpu.VMEM((ch,), jnp.int32),
               pltpu.VMEM((ch,), jnp.int32),
               pltpu.VMEM((ch,), jnp.int32),
               pltpu.VMEM((ch, dh), jnp.float32),
               pltpu.VMEM((ch, dh), jnp.float32),
               pltpu.VMEM((16, dh), jnp.float32),
               pltpu.SemaphoreType.DMA,
               pltpu.SemaphoreType.DMA,
               pltpu.SemaphoreType.DMA,
               pltpu.SemaphoreType.DMA]
    if weighted:
        scratch += [pltpu.VMEM((ch,), jnp.float32),
                    pltpu.VMEM((ch,), jnp.float32),
                    pltpu.VMEM((t_pw,), jnp.float32),
                    pltpu.VMEM((mp,), jnp.float32)]
    if has_bias:
        scratch.append(pltpu.VMEM((dh,), jnp.float32))
    scratch.append(pltpu.VMEM_SHARED((mp, dh), jnp.float32))

    def body(*args):
        if weighted:
            if has_bias:
                (tab_h, src2_h, dst_h, ex_h, ssum_h, bias_h), rest = args[:6], args[6:]
            else:
                (tab_h, src2_h, dst_h, ex_h, ssum_h), rest = args[:5], args[5:]
                bias_h = None
        else:
            if has_bias:
                (tab_h, src2_h, dst_h, bias_h), rest = args[:4], args[4:]
            else:
                (tab_h, src2_h, dst_h), rest = args[:3], args[3:]
                bias_h = None
            ex_h = ssum_h = None
        if emit_a:
            out_h, a_h = rest[0], rest[1]
            rest = rest[2:]
        else:
            out_h = rest[0]
            a_h = None
            rest = rest[1:]
        (sidx0, sidx1, didx0, didx1, rows0, rows1, zbuf,
         smi0, smi1, smr0, smr1) = rest[:11]
        rest = rest[11:]
        eb0 = eb1 = abuf = sv = biasv = None
        if weighted:
            eb0, eb1, abuf, sv = rest[:4]
            rest = rest[4:]
        if has_bias:
            biasv = rest[0]
            rest = rest[1:]
        acc = rest[0]

        c = lax.axis_index("c")
        s = lax.axis_index("s")

        zero16 = jnp.zeros((_L,), jnp.float32)
        if has_bias and edge_split:
            pltpu.sync_copy(bias_h, biasv)
            bvals = [jnp.where(c == 0, biasv[pl.ds(j * _L, _L)], zero16)
                     for j in range(dh // _L)]
        elif has_bias:
            pltpu.sync_copy(bias_h.at[pl.ds(c * dh, dh)], biasv)
            bvals = [biasv[pl.ds(j * _L, _L)] for j in range(dh // _L)]
        else:
            bvals = [zero16] * (dh // _L)

        def zrow(i, _):
            for j in range(dh // _L):
                zbuf[i, pl.ds(j * _L, _L)] = bvals[j]
            return 0

        lax.fori_loop(0, 16, zrow, 0)
        for k in range(rpt // 16):
            pltpu.sync_copy(zbuf, acc.at[pl.ds(s * rpt + k * 16, 16)])
        plsc.subcore_barrier()

        if weighted:
            pltpu.sync_copy(ssum_h, sv)

        base = c * (ep // 2) + s * t_pw if edge_split else s * t_pw

        def idx_start(g, sb, db, eb, sm):
            off = base + g * ch
            pltpu.async_copy(src2_h.at[pl.ds(c * ep + off, ch)], sb, sm)
            pltpu.async_copy(dst_h.at[pl.ds(off, ch)], db, sm)
            if weighted:
                pltpu.async_copy(ex_h.at[pl.ds(off, ch)], eb, sm)

        def idx_wait(g, sb, db, eb, sm):
            off = base + g * ch
            pltpu.make_async_copy(src2_h.at[pl.ds(c * ep + off, ch)], sb, sm).wait()
            pltpu.make_async_copy(dst_h.at[pl.ds(off, ch)], db, sm).wait()
            if weighted:
                pltpu.make_async_copy(ex_h.at[pl.ds(off, ch)], eb, sm).wait()

        def gather_start(sb, rows, sm):
            pltpu.async_copy(tab_h.at[sb], rows, sm)

        def gather_wait(sb, rows, sm):
            pltpu.make_async_copy(tab_h.at[sb], rows, sm).wait()

        def process(g, db, eb, rows):
            if weighted:
                for j in range(ch // _L):
                    dv = db[pl.ds(j * _L, _L)]
                    aw = eb[pl.ds(j * _L, _L)] / plsc.load_gather(sv, [dv])
                    abuf[pl.ds(g * ch + j * _L, _L)] = aw

                def escale(e, _):
                    aw = plsc.load_gather(
                        abuf, [jnp.full((_L,), 0, jnp.int32) + (g * ch + e)])
                    scaled = [rows[e, pl.ds(j * _L, _L)] * aw
                              for j in range(dh // _L)]
                    for j in range(dh // _L):
                        rows[e, pl.ds(j * _L, _L)] = scaled[j]
                    return 0

                lax.fori_loop(0, ch, escale, 0)
            pltpu.sync_copy(rows, acc.at[db], add=True)

        idx_start(0, sidx0, didx0, eb0, smi0)
        idx_wait(0, sidx0, didx0, eb0, smi0)
        gather_start(sidx0, rows0, smr0)
        idx_start(1, sidx1, didx1, eb1, smi1)

        def pair(gp, _):
            g0 = 2 * gp
            gather_wait(sidx0, rows0, smr0)
            idx_wait(g0 + 1, sidx1, didx1, eb1, smi1)
            gather_start(sidx1, rows1, smr1)
            process(g0, didx0, eb0, rows0)

            @pl.when(g0 + 2 < nch)
            def _():
                idx_start(g0 + 2, sidx0, didx0, eb0, smi0)

            gather_wait(sidx1, rows1, smr1)

            @pl.when(g0 + 2 < nch)
            def _():
                idx_wait(g0 + 2, sidx0, didx0, eb0, smi0)
                gather_start(sidx0, rows0, smr0)

            process(g0 + 1, didx1, eb1, rows1)

            @pl.when(g0 + 3 < nch)
            def _():
                idx_start(g0 + 3, sidx1, didx1, eb1, smi1)

            return 0

        lax.fori_loop(0, nch // 2, pair, 0)

        if weighted and emit_a and edge_split:
            pltpu.sync_copy(abuf, a_h.at[pl.ds(base, t_pw)])
        elif weighted and emit_a:
            @pl.when(c == 0)
            def _():
                pltpu.sync_copy(abuf, a_h.at[pl.ds(base, t_pw)])
        plsc.subcore_barrier()
        for k in range(rpt // 128):
            r0 = s * rpt + k * 128
            pltpu.sync_copy(acc.at[pl.ds(r0, 128)],
                            out_h.at[pl.ds(c * mp + r0, 128)])

    fn = pl.kernel(body, out_type=out_types, mesh=_sc_mesh(),
                   compiler_params=_SC_PARAMS, scratch_types=scratch)
    ins = [tab, src2.reshape(-1), dst]
    if weighted:
        ins += [ex, s_sum]
    if has_bias:
        ins.append(bias)
    res = fn(*ins)
    return res if emit_a else res[0]


def _sc_edge_softmax_num(el, er, src, dst):
    mp = el.shape[0]
    ep = src.shape[0]
    e_pw = ep // _NW

    def body(el_h, er_h, src_h, dst_h, ex_h, hist_h, elv, erv, sbuf, dbuf,
             exbuf, hist):
        c = lax.axis_index("c")
        s = lax.axis_index("s")
        w = s * _NC + c

        def zero_body(i, _):
            hist[pl.ds(i * _L, _L)] = jnp.zeros((_L,), jnp.float32)
            return 0

        lax.fori_loop(0, mp // _L, zero_body, 0)
        pltpu.sync_copy(el_h, elv)
        pltpu.sync_copy(er_h, erv)
        pltpu.sync_copy(src_h.at[pl.ds(w * e_pw, e_pw)], sbuf)
        pltpu.sync_copy(dst_h.at[pl.ds(w * e_pw, e_pw)], dbuf)

        def body_i(i, _):
            sv = sbuf[pl.ds(i * _L, _L)]
            dv = dbuf[pl.ds(i * _L, _L)]
            x = plsc.load_gather(elv, [sv]) + plsc.load_gather(erv, [dv])
            x = jnp.maximum(x, 0.2 * x)
            exv = jnp.exp(x)
            exbuf[pl.ds(i * _L, _L)] = exv
            plsc.addupdate_scatter(hist, [dv], exv)
            return 0

        lax.fori_loop(0, e_pw // _L, body_i, 0)
        pltpu.sync_copy(exbuf, ex_h.at[pl.ds(w * e_pw, e_pw)])
        pltpu.sync_copy(hist, hist_h.at[w])

    fn = pl.kernel(
        body,
        out_type=[jax.ShapeDtypeStruct((ep,), jnp.float32),
                  jax.ShapeDtypeStruct((_NW, mp), jnp.float32)],
        mesh=_sc_mesh(),
        compiler_params=_SC_PARAMS,
        scratch_types=[pltpu.VMEM((mp,), jnp.float32),
                       pltpu.VMEM((mp,), jnp.float32),
                       pltpu.VMEM((e_pw,), jnp.int32),
                       pltpu.VMEM((e_pw,), jnp.int32),
                       pltpu.VMEM((e_pw,), jnp.float32),
                       pltpu.VMEM((mp,), jnp.float32)])
    return fn(el, er, src, dst)


def _sc_segmax_relu(tab, src, dst):
    mp, d = tab.shape
    ep = src.shape[0]
    rpw = mp // _NW
    ce = 2048
    nst = ep // ce
    ndj = d // _L

    def body(tab_h, src_h, dst_h, out_h, sbuf0, dbuf0, sbuf1, dbuf1, msrc,
             mdl, grows, acc, sem, semA, semB):
        c = lax.axis_index("c")
        s = lax.axis_index("s")
        w = s * _NC + c
        lo = w * rpw

        def zrow(i, _):
            for j in range(ndj):
                acc[i, pl.ds(j * _L, _L)] = jnp.zeros((_L,), jnp.float32)
            return 0

        lax.fori_loop(0, rpw, zrow, 0)

        def zidx(i, _):
            msrc[pl.ds(i * _L, _L)] = jnp.zeros((_L,), jnp.int32)
            mdl[pl.ds(i * _L, _L)] = jnp.zeros((_L,), jnp.int32)
            return 0

        lax.fori_loop(0, ce // _L, zidx, 0)

        cols = [lax.iota(jnp.int32, _L) + j * _L for j in range(ndj)]

        def stage_start(st, sb, db, sm):
            soff = st * ce
            pltpu.async_copy(src_h.at[pl.ds(soff, ce)], sb, sm)
            pltpu.async_copy(dst_h.at[pl.ds(soff, ce)], db, sm)

        def stage(st, sbuf, dbuf, sm):
            soff = st * ce
            pltpu.make_async_copy(src_h.at[pl.ds(soff, ce)], sbuf, sm).wait()
            pltpu.make_async_copy(dst_h.at[pl.ds(soff, ce)], dbuf, sm).wait()

            def filt(j, cnt):
                sv = sbuf[pl.ds(j * _L, _L)]
                dl = dbuf[pl.ds(j * _L, _L)] - lo
                mask = (dl >= 0) & (dl < rpw)
                pos = cnt + plsc.cumsum(jnp.where(mask, 1, 0)) - 1
                plsc.store_scatter(msrc, [pos], sv, mask=mask)
                plsc.store_scatter(mdl, [pos], dl, mask=mask)
                return cnt + plsc.all_reduce_population_count(mask)

            cntv = lax.fori_loop(0, ce // _L, filt,
                                 jnp.zeros((_L,), jnp.int32))
            cnt = jnp.max(cntv)

            def group(gi, _):
                gb = gi * 64
                pltpu.async_copy(tab_h.at[msrc.at[pl.ds(gb, 64)]], grows,
                                 sem).wait()

                nloc = jnp.minimum(64, cnt - gb)

                def upd(e, _):
                    dlw = plsc.load_gather(
                        mdl, [jnp.full((_L,), 0, jnp.int32) + (gb + e)])
                    news = [jnp.maximum(plsc.load_gather(acc, [dlw, cols[j]]),
                                        grows[e, pl.ds(j * _L, _L)])
                            for j in range(ndj)]
                    for j in range(ndj):
                        plsc.store_scatter(acc, [dlw, cols[j]], news[j])
                    return 0

                lax.fori_loop(0, nloc, upd, 0)
                return 0

            lax.fori_loop(0, (cnt + 63) // 64, group, 0)

        stage_start(0, sbuf0, dbuf0, semA)

        def spair(sp, _):
            st0 = 2 * sp
            stage_start(st0 + 1, sbuf1, dbuf1, semB)
            stage(st0, sbuf0, dbuf0, semA)

            @pl.when(sp + 1 < nst // 2)
            def _():
                stage_start(st0 + 2, sbuf0, dbuf0, semA)

            stage(st0 + 1, sbuf1, dbuf1, semB)
            return 0

        lax.fori_loop(0, nst // 2, spair, 0)
        pltpu.sync_copy(acc, out_h.at[pl.ds(lo, rpw)])

    fn = pl.kernel(
        body,
        out_type=jax.ShapeDtypeStruct((mp, d), jnp.float32),
        mesh=_sc_mesh(),
        compiler_params=_SC_PARAMS,
        scratch_types=[pltpu.VMEM((ce,), jnp.int32),
                       pltpu.VMEM((ce,), jnp.int32),
                       pltpu.VMEM((ce,), jnp.int32),
                       pltpu.VMEM((ce,), jnp.int32),
                       pltpu.VMEM((ce,), jnp.int32),
                       pltpu.VMEM((ce,), jnp.int32),
                       pltpu.VMEM((64, d), jnp.float32),
                       pltpu.VMEM((rpw, d), jnp.float32),
                       pltpu.SemaphoreType.DMA,
                       pltpu.SemaphoreType.DMA,
                       pltpu.SemaphoreType.DMA])
    return fn(tab, src, dst)


def _add2_body(a_ref, b_ref, o_ref):
    o_ref[...] = a_ref[...] + b_ref[...]


def _add2(a, b):
    m, d = a.shape
    return pl.pallas_call(
        _add2_body,
        grid=(m // _BM,),
        in_specs=[pl.BlockSpec((_BM, d), lambda i: (i, 0)),
                  pl.BlockSpec((_BM, d), lambda i: (i, 0))],
        out_specs=pl.BlockSpec((_BM, d), lambda i: (i, 0)),
        out_shape=jax.ShapeDtypeStruct((m, d), jnp.float32),
    )(a, b)


def _colsum_body(h_ref, o_ref):
    o_ref[...] = jnp.sum(h_ref[...], axis=0)


def _colsum(h):
    nw, m = h.shape
    return pl.pallas_call(
        _colsum_body,
        grid=(m // _BM,),
        in_specs=[pl.BlockSpec((nw, _BM), lambda i: (0, i))],
        out_specs=pl.BlockSpec((_BM,), lambda i: (i,)),
        out_shape=jax.ShapeDtypeStruct((m,), jnp.float32),
    )(h)


def _mm_body(a_ref, w_ref, scale_ref, bias_ref, attn_ref, out_ref, e_ref,
             *, has_scale, has_bias, has_attn):
    a = a_ref[...]
    if has_scale:
        s = lax.rsqrt(jnp.clip(jnp.sum(scale_ref[...], axis=0), 1.0, None))
        a = a * s[:, None]
    acc = jnp.dot(a, w_ref[...], preferred_element_type=jnp.float32)
    if has_bias:
        acc = acc + bias_ref[...][None, :]
    out_ref[...] = acc
    if has_attn:
        e_ref[...] = jnp.sum(acc * attn_ref[...][None, :], axis=1)


def _mm(a, w, *, scale=None, bias=None, attn=None):
    m, k = a.shape
    d = w.shape[1]
    has_scale = scale is not None
    has_bias = bias is not None
    has_attn = attn is not None
    grid = (m // _BM,)
    in_specs = [
        pl.BlockSpec((_BM, k), lambda i: (i, 0)),
        pl.BlockSpec((k, d), lambda i: (0, 0)),
        pl.BlockSpec((_NW, _BM), lambda i: (0, i)) if has_scale else pl.BlockSpec((128,), lambda i: (0,)),
        pl.BlockSpec((d,), lambda i: (0,)) if has_bias else pl.BlockSpec((128,), lambda i: (0,)),
        pl.BlockSpec((d,), lambda i: (0,)) if has_attn else pl.BlockSpec((128,), lambda i: (0,)),
    ]
    out_shapes = [jax.ShapeDtypeStruct((m, d), jnp.float32),
                  jax.ShapeDtypeStruct((m,), jnp.float32)]
    out_specs = [pl.BlockSpec((_BM, d), lambda i: (i, 0)),
                 pl.BlockSpec((_BM,), lambda i: (i,))]
    dummy = jnp.zeros((128,), jnp.float32)
    out, e = pl.pallas_call(
        functools.partial(_mm_body, has_scale=has_scale, has_bias=has_bias,
                          has_attn=has_attn),
        grid=grid,
        in_specs=in_specs,
        out_specs=out_specs,
        out_shape=out_shapes,
    )(a, w,
      scale if has_scale else dummy,
      bias if has_bias else dummy,
      attn if has_attn else dummy)
    if has_attn:
        return out, e
    return out


def _scale_bias_body(z_ref, s_ref, b_ref, o_ref):
    s = lax.rsqrt(jnp.clip(jnp.sum(s_ref[...], axis=0), 1.0, None))
    o_ref[...] = z_ref[...] * s[:, None] + b_ref[...][None, :]


def _scale_bias(z, s, b):
    m, d = z.shape
    return pl.pallas_call(
        _scale_bias_body,
        grid=(m // _BM,),
        in_specs=[pl.BlockSpec((_BM, d), lambda i: (i, 0)),
                  pl.BlockSpec((_NW, _BM), lambda i: (0, i)),
                  pl.BlockSpec((d,), lambda i: (0,))],
        out_specs=pl.BlockSpec((_BM, d), lambda i: (i, 0)),
        out_shape=jax.ShapeDtypeStruct((m, d), jnp.float32),
    )(z, s, b)


def _edge_softmax(e, dst, num_dst):
    m = jax.ops.segment_max(e, dst, num_segments=num_dst)
    m = jnp.where(jnp.isfinite(m), m, 0.0)
    ex = jnp.exp(e - m[dst])
    s = jax.ops.segment_sum(ex, dst, num_segments=num_dst)
    return ex / s[dst]


def kernel(feat_N, feat_Q, edge_nn, edge_np, edge_pq, W_gcn, b_gcn, W_src1,
           W_dst1, attn_l1, attn_r1, bias1, W_src2, W_dst2, attn_l2, attn_r2,
           bias2):
    n_n = feat_N.shape[0]
    mp = ((n_n + _BM - 1) // _BM) * _BM
    pad = mp - n_n
    featN_p = jnp.pad(feat_N, ((0, pad), (0, 0)))
    featQ_p = jnp.pad(feat_Q, ((0, pad), (0, 0)))

    def pad_edges(e2):
        s, d = e2[0], e2[1]
        epad = (-s.shape[0]) % (_NW * 128)
        sp = jnp.pad(s, (0, epad), constant_values=n_n)
        dp = jnp.pad(d, (0, epad), constant_values=mp - 1)
        return sp, dp, jnp.stack([sp, sp + mp])

    def split2(x):
        dh = x.shape[1] // 2
        return jnp.concatenate([x[:, :dh], x[:, dh:]], axis=0)

    def join2(x):
        mp_ = x.shape[0] // 2
        return jnp.concatenate([x[:mp_], x[mp_:]], axis=1)

    src_p, dst_p, src2 = pad_edges(edge_nn)
    hs_, hd_ = _sc_degree_hists(src_p, dst_p, mp)
    y = _mm(featN_p, W_gcn, scale=hs_)
    z = join2(_sc_gather_segsum(split2(y), src2, dst_p))
    h = _scale_bias(z, hd_, b_gcn)

    s1p, d1p, s1_2 = pad_edges(edge_np)
    hp = _sc_segmax_relu(h, s1p, d1p)

    fs, el = _mm(h, W_src1, attn=attn_l1)
    _fd, er = _mm(hp, W_dst1, attn=attn_r1)
    ex1, sh1 = _sc_edge_softmax_num(el, er, s1p, d1p)
    hp2 = join2(_sc_gather_segsum(split2(fs), s1_2, d1p, ex=ex1,
                                  s_sum=_colsum(sh1), bias=bias1))

    s2p, d2p, s2_2 = pad_edges(edge_pq)
    fs2, el2 = _mm(hp2, W_src2, attn=attn_l2)
    _fd2, er2 = _mm(featQ_p, W_dst2, attn=attn_r2)
    ex2, sh2 = _sc_edge_softmax_num(el2, er2, s2p, d2p)
    outf, a2p = _sc_gather_segsum(fs2, jnp.stack([s2p, s2p]), d2p, ex=ex2,
                                  s_sum=_colsum(sh2), bias=bias2, emit_a=True,
                                  edge_split=True)
    out = _add2(outf[:mp], outf[mp:])
    return out[:n_n], a2p[:edge_pq.shape[1]]


# speedup vs baseline: 5.0255x; 1.8755x over previous
import functools

import jax
import jax.numpy as jnp
from jax import lax
from jax.experimental import pallas as pl
from jax.experimental.pallas import tpu as pltpu
from jax.experimental.pallas import tpu_sc as plsc

_BM = 1024
_NC, _NS, _L = 2, 16, 16
_NW = _NC * _NS

_SC_PARAMS = pltpu.CompilerParams(needs_layout_passes=False)


@functools.cache
def _sc_mesh():
    return plsc.VectorSubcoreMesh(core_axis_name="c", subcore_axis_name="s",
                                  num_cores=_NC, num_subcores=_NS)


def _sc_degree_hists(src, dst, mp):
    e_pw = src.shape[0] // _NW

    @functools.partial(
        pl.kernel,
        out_type=[jax.ShapeDtypeStruct((_NW, mp), jnp.float32),
                  jax.ShapeDtypeStruct((_NW, mp), jnp.float32)],
        mesh=_sc_mesh(),
        compiler_params=_SC_PARAMS,
        scratch_types=[pltpu.VMEM((e_pw,), jnp.int32),
                       pltpu.VMEM((e_pw,), jnp.int32),
                       pltpu.VMEM((mp,), jnp.float32),
                       pltpu.VMEM((mp,), jnp.float32)],
    )
    def deg_kernel(src_hbm, dst_hbm, osrc_hbm, odst_hbm, sbuf, dbuf, hs, hd):
        c = lax.axis_index("c")
        s = lax.axis_index("s")
        w = s * _NC + c

        def zero_body(i, _):
            z = jnp.zeros((_L,), jnp.float32)
            hs[pl.ds(i * _L, _L)] = z
            hd[pl.ds(i * _L, _L)] = z
            return 0

        lax.fori_loop(0, mp // _L, zero_body, 0)
        pltpu.sync_copy(src_hbm.at[pl.ds(w * e_pw, e_pw)], sbuf)
        pltpu.sync_copy(dst_hbm.at[pl.ds(w * e_pw, e_pw)], dbuf)
        ones = jnp.full((_L,), 1.0, jnp.float32)

        def body(i, _):
            sv = sbuf[pl.ds(i * _L, _L)]
            dv = dbuf[pl.ds(i * _L, _L)]
            plsc.addupdate_scatter(hs, [sv], ones)
            plsc.addupdate_scatter(hd, [dv], ones)
            return 0

        lax.fori_loop(0, e_pw // _L, body, 0)
        pltpu.sync_copy(hs, osrc_hbm.at[w])
        pltpu.sync_copy(hd, odst_hbm.at[w])

    return deg_kernel(src, dst)


def _sc_gather_segsum(tab, src2, dst, ex=None, s_sum=None, bias=None,
                      emit_a=False, edge_split=False):
    if edge_split:
        mp, dh = tab.shape
        two_mp = 2 * mp
    else:
        two_mp, dh = tab.shape
        mp = two_mp // 2
    ep = dst.shape[0]
    t_pw = ep // _NW if edge_split else ep // _NS
    nch = t_pw // 128
    rpt = mp // _NS
    weighted = ex is not None
    has_bias = bias is not None

    ch = 64
    nch = t_pw // ch
    out_types = [jax.ShapeDtypeStruct((two_mp, dh), jnp.float32)]
    if emit_a:
        out_types.append(jax.ShapeDtypeStruct((ep,), jnp.float32))
    scratch = [pltpu.VMEM((ch,), jnp.int32),
               pltpu.VMEM((ch,), jnp.int32),
               pltpu.VMEM((ch,), jnp.int32),
               pltpu.VMEM((ch,), jnp.int32),
               pltpu.VMEM((ch, dh), jnp.float32),
               pltpu.VMEM((ch, dh), jnp.float32),
               pltpu.VMEM((16, dh), jnp.float32),
               pltpu.SemaphoreType.DMA,
               pltpu.SemaphoreType.DMA,
               pltpu.SemaphoreType.DMA,
               pltpu.SemaphoreType.DMA]
    if weighted:
        scratch += [pltpu.VMEM((ch,), jnp.float32),
                    pltpu.VMEM((ch,), jnp.float32),
                    pltpu.VMEM((t_pw,), jnp.float32),
                    pltpu.VMEM((mp,), jnp.float32)]
    if has_bias:
        scratch.append(pltpu.VMEM((dh,), jnp.float32))
    scratch.append(pltpu.VMEM_SHARED((mp, dh), jnp.float32))

    def body(*args):
        if weighted:
            if has_bias:
                (tab_h, src2_h, dst_h, ex_h, ssum_h, bias_h), rest = args[:6], args[6:]
            else:
                (tab_h, src2_h, dst_h, ex_h, ssum_h), rest = args[:5], args[5:]
                bias_h = None
        else:
            if has_bias:
                (tab_h, src2_h, dst_h, bias_h), rest = args[:4], args[4:]
            else:
                (tab_h, src2_h, dst_h), rest = args[:3], args[3:]
                bias_h = None
            ex_h = ssum_h = None
        if emit_a:
            out_h, a_h = rest[0], rest[1]
            rest = rest[2:]
        else:
            out_h = rest[0]
            a_h = None
            rest = rest[1:]
        (sidx0, sidx1, didx0, didx1, rows0, rows1, zbuf,
         smi0, smi1, smr0, smr1) = rest[:11]
        rest = rest[11:]
        eb0 = eb1 = abuf = sv = biasv = None
        if weighted:
            eb0, eb1, abuf, sv = rest[:4]
            rest = rest[4:]
        if has_bias:
            biasv = rest[0]
            rest = rest[1:]
        acc = rest[0]

        c = lax.axis_index("c")
        s = lax.axis_index("s")

        zero16 = jnp.zeros((_L,), jnp.float32)
        if has_bias and edge_split:
            pltpu.sync_copy(bias_h, biasv)
            bvals = [jnp.where(c == 0, biasv[pl.ds(j * _L, _L)], zero16)
                     for j in range(dh // _L)]
        elif has_bias:
            pltpu.sync_copy(bias_h.at[pl.ds(c * dh, dh)], biasv)
            bvals = [biasv[pl.ds(j * _L, _L)] for j in range(dh // _L)]
        else:
            bvals = [zero16] * (dh // _L)

        def zrow(i, _):
            for j in range(dh // _L):
                zbuf[i, pl.ds(j * _L, _L)] = bvals[j]
            return 0

        lax.fori_loop(0, 16, zrow, 0)
        for k in range(rpt // 16):
            pltpu.sync_copy(zbuf, acc.at[pl.ds(s * rpt + k * 16, 16)])
        plsc.subcore_barrier()

        if weighted:
            pltpu.sync_copy(ssum_h, sv)

        base = c * (ep // 2) + s * t_pw if edge_split else s * t_pw

        def idx_start(g, sb, db, eb, sm):
            off = base + g * ch
            pltpu.async_copy(src2_h.at[pl.ds(c * ep + off, ch)], sb, sm)
            pltpu.async_copy(dst_h.at[pl.ds(off, ch)], db, sm)
            if weighted:
                pltpu.async_copy(ex_h.at[pl.ds(off, ch)], eb, sm)

        def idx_wait(g, sb, db, eb, sm):
            off = base + g * ch
            pltpu.make_async_copy(src2_h.at[pl.ds(c * ep + off, ch)], sb, sm).wait()
            pltpu.make_async_copy(dst_h.at[pl.ds(off, ch)], db, sm).wait()
            if weighted:
                pltpu.make_async_copy(ex_h.at[pl.ds(off, ch)], eb, sm).wait()

        def gather_start(sb, rows, sm):
            pltpu.async_copy(tab_h.at[sb], rows, sm)

        def gather_wait(sb, rows, sm):
            pltpu.make_async_copy(tab_h.at[sb], rows, sm).wait()

        def process(g, db, eb, rows):
            if weighted:
                for j in range(ch // _L):
                    dv = db[pl.ds(j * _L, _L)]
                    aw = eb[pl.ds(j * _L, _L)] / plsc.load_gather(sv, [dv])
                    abuf[pl.ds(g * ch + j * _L, _L)] = aw

                def escale(e, _):
                    aw = plsc.load_gather(
                        abuf, [jnp.full((_L,), 0, jnp.int32) + (g * ch + e)])
                    scaled = [rows[e, pl.ds(j * _L, _L)] * aw
                              for j in range(dh // _L)]
                    for j in range(dh // _L):
                        rows[e, pl.ds(j * _L, _L)] = scaled[j]
                    return 0

                lax.fori_loop(0, ch, escale, 0)
            pltpu.sync_copy(rows, acc.at[db], add=True)

        idx_start(0, sidx0, didx0, eb0, smi0)
        idx_wait(0, sidx0, didx0, eb0, smi0)
        gather_start(sidx0, rows0, smr0)
        idx_start(1, sidx1, didx1, eb1, smi1)

        def pair(gp, _):
            g0 = 2 * gp
            gather_wait(sidx0, rows0, smr0)
            idx_wait(g0 + 1, sidx1, didx1, eb1, smi1)
            gather_start(sidx1, rows1, smr1)
            process(g0, didx0, eb0, rows0)

            @pl.when(g0 + 2 < nch)
            def _():
                idx_start(g0 + 2, sidx0, didx0, eb0, smi0)

            gather_wait(sidx1, rows1, smr1)

            @pl.when(g0 + 2 < nch)
            def _():
                idx_wait(g0 + 2, sidx0, didx0, eb0, smi0)
                gather_start(sidx0, rows0, smr0)

            process(g0 + 1, didx1, eb1, rows1)

            @pl.when(g0 + 3 < nch)
            def _():
                idx_start(g0 + 3, sidx1, didx1, eb1, smi1)

            return 0

        lax.fori_loop(0, nch // 2, pair, 0)

        if weighted and emit_a and edge_split:
            pltpu.sync_copy(abuf, a_h.at[pl.ds(base, t_pw)])
        elif weighted and emit_a:
            @pl.when(c == 0)
            def _():
                pltpu.sync_copy(abuf, a_h.at[pl.ds(base, t_pw)])
        plsc.subcore_barrier()
        for k in range(rpt // 128):
            r0 = s * rpt + k * 128
            pltpu.sync_copy(acc.at[pl.ds(r0, 128)],
                            out_h.at[pl.ds(c * mp + r0, 128)])

    fn = pl.kernel(body, out_type=out_types, mesh=_sc_mesh(),
                   compiler_params=_SC_PARAMS, scratch_types=scratch)
    ins = [tab, src2.reshape(-1), dst]
    if weighted:
        ins += [ex, s_sum]
    if has_bias:
        ins.append(bias)
    res = fn(*ins)
    return res if emit_a else res[0]


def _sc_edge_softmax_num(el, er, src, dst):
    mp = el.shape[0]
    ep = src.shape[0]
    e_pw = ep // _NW

    def body(el_h, er_h, src_h, dst_h, ex_h, hist_h, elv, erv, sbuf, dbuf,
             exbuf, hist):
        c = lax.axis_index("c")
        s = lax.axis_index("s")
        w = s * _NC + c

        def zero_body(i, _):
            hist[pl.ds(i * _L, _L)] = jnp.zeros((_L,), jnp.float32)
            return 0

        lax.fori_loop(0, mp // _L, zero_body, 0)
        pltpu.sync_copy(el_h, elv)
        pltpu.sync_copy(er_h, erv)
        pltpu.sync_copy(src_h.at[pl.ds(w * e_pw, e_pw)], sbuf)
        pltpu.sync_copy(dst_h.at[pl.ds(w * e_pw, e_pw)], dbuf)

        def body_i(i, _):
            sv = sbuf[pl.ds(i * _L, _L)]
            dv = dbuf[pl.ds(i * _L, _L)]
            x = plsc.load_gather(elv, [sv]) + plsc.load_gather(erv, [dv])
            x = jnp.maximum(x, 0.2 * x)
            exv = jnp.exp(x)
            exbuf[pl.ds(i * _L, _L)] = exv
            plsc.addupdate_scatter(hist, [dv], exv)
            return 0

        lax.fori_loop(0, e_pw // _L, body_i, 0)
        pltpu.sync_copy(exbuf, ex_h.at[pl.ds(w * e_pw, e_pw)])
        pltpu.sync_copy(hist, hist_h.at[w])

    fn = pl.kernel(
        body,
        out_type=[jax.ShapeDtypeStruct((ep,), jnp.float32),
                  jax.ShapeDtypeStruct((_NW, mp), jnp.float32)],
        mesh=_sc_mesh(),
        compiler_params=_SC_PARAMS,
        scratch_types=[pltpu.VMEM((mp,), jnp.float32),
                       pltpu.VMEM((mp,), jnp.float32),
                       pltpu.VMEM((e_pw,), jnp.int32),
                       pltpu.VMEM((e_pw,), jnp.int32),
                       pltpu.VMEM((e_pw,), jnp.float32),
                       pltpu.VMEM((mp,), jnp.float32)])
    return fn(el, er, src, dst)


def _sc_segmax_relu(tab, src, dst):
    mp, d = tab.shape
    ep = src.shape[0]
    rpw = mp // _NW
    ce = 2048
    nst = ep // ce
    ndj = d // _L

    def body(tab_h, src_h, dst_h, out_h, sbuf0, dbuf0, sbuf1, dbuf1, msrc,
             mdl, grows0, grows1, acc, semG0, semG1, semA, semB):
        c = lax.axis_index("c")
        s = lax.axis_index("s")
        w = s * _NC + c
        lo = w * rpw

        def zrow(i, _):
            for j in range(ndj):
                acc[i, pl.ds(j * _L, _L)] = jnp.zeros((_L,), jnp.float32)
            return 0

        lax.fori_loop(0, rpw, zrow, 0)

        def zidx(i, _):
            msrc[pl.ds(i * _L, _L)] = jnp.zeros((_L,), jnp.int32)
            mdl[pl.ds(i * _L, _L)] = jnp.zeros((_L,), jnp.int32)
            return 0

        lax.fori_loop(0, ce // _L, zidx, 0)

        cols = [lax.iota(jnp.int32, _L) + j * _L for j in range(ndj)]

        def stage_start(st, sb, db, sm):
            soff = st * ce
            pltpu.async_copy(src_h.at[pl.ds(soff, ce)], sb, sm)
            pltpu.async_copy(dst_h.at[pl.ds(soff, ce)], db, sm)

        def stage(st, sbuf, dbuf, sm):
            soff = st * ce
            pltpu.make_async_copy(src_h.at[pl.ds(soff, ce)], sbuf, sm).wait()
            pltpu.make_async_copy(dst_h.at[pl.ds(soff, ce)], dbuf, sm).wait()

            def filt(j, cnt):
                sv = sbuf[pl.ds(j * _L, _L)]
                dl = dbuf[pl.ds(j * _L, _L)] - lo
                mask = (dl >= 0) & (dl < rpw)
                pos = cnt + plsc.cumsum(jnp.where(mask, 1, 0)) - 1
                plsc.store_scatter(msrc, [pos], sv, mask=mask)
                plsc.store_scatter(mdl, [pos], dl, mask=mask)
                return cnt + plsc.all_reduce_population_count(mask)

            cntv = lax.fori_loop(0, ce // _L, filt,
                                 jnp.zeros((_L,), jnp.int32))
            cnt = jnp.max(cntv)

            gr = 32
            ng = (cnt + gr - 1) // gr

            def g_start(gi, grows, sm):
                pltpu.async_copy(tab_h.at[msrc.at[pl.ds(gi * gr, gr)]],
                                 grows, sm)

            def g_wait(gi, grows, sm):
                pltpu.make_async_copy(tab_h.at[msrc.at[pl.ds(gi * gr, gr)]],
                                      grows, sm).wait()

            def g_upd(gi, grows):
                gb = gi * gr
                nloc = jnp.minimum(gr, cnt - gb)

                def upd(e, _):
                    dlw = plsc.load_gather(
                        mdl, [jnp.full((_L,), 0, jnp.int32) + (gb + e)])
                    news = [jnp.maximum(plsc.load_gather(acc, [dlw, cols[j]]),
                                        grows[e, pl.ds(j * _L, _L)])
                            for j in range(ndj)]
                    for j in range(ndj):
                        plsc.store_scatter(acc, [dlw, cols[j]], news[j])
                    return 0

                lax.fori_loop(0, nloc, upd, 0)

            @pl.when(ng > 0)
            def _():
                g_start(0, grows0, semG0)

            def gpair(q, _):
                g0 = 2 * q

                @pl.when(g0 + 1 < ng)
                def _():
                    g_start(g0 + 1, grows1, semG1)

                g_wait(g0, grows0, semG0)
                g_upd(g0, grows0)

                @pl.when(g0 + 2 < ng)
                def _():
                    g_start(g0 + 2, grows0, semG0)

                @pl.when(g0 + 1 < ng)
                def _():
                    g_wait(g0 + 1, grows1, semG1)
                    g_upd(g0 + 1, grows1)

                return 0

            lax.fori_loop(0, (ng + 1) // 2, gpair, 0)

        stage_start(0, sbuf0, dbuf0, semA)

        def spair(sp, _):
            st0 = 2 * sp
            stage_start(st0 + 1, sbuf1, dbuf1, semB)
            stage(st0, sbuf0, dbuf0, semA)

            @pl.when(sp + 1 < nst // 2)
            def _():
                stage_start(st0 + 2, sbuf0, dbuf0, semA)

            stage(st0 + 1, sbuf1, dbuf1, semB)
            return 0

        lax.fori_loop(0, nst // 2, spair, 0)
        pltpu.sync_copy(acc, out_h.at[pl.ds(lo, rpw)])

    fn = pl.kernel(
        body,
        out_type=jax.ShapeDtypeStruct((mp, d), jnp.float32),
        mesh=_sc_mesh(),
        compiler_params=_SC_PARAMS,
        scratch_types=[pltpu.VMEM((ce,), jnp.int32),
                       pltpu.VMEM((ce,), jnp.int32),
                       pltpu.VMEM((ce,), jnp.int32),
                       pltpu.VMEM((ce,), jnp.int32),
                       pltpu.VMEM((ce,), jnp.int32),
                       pltpu.VMEM((ce,), jnp.int32),
                       pltpu.VMEM((32, d), jnp.float32),
                       pltpu.VMEM((32, d), jnp.float32),
                       pltpu.VMEM((rpw, d), jnp.float32),
                       pltpu.SemaphoreType.DMA,
                       pltpu.SemaphoreType.DMA,
                       pltpu.SemaphoreType.DMA,
                       pltpu.SemaphoreType.DMA])
    return fn(tab, src, dst)


def _add2_body(a_ref, b_ref, o_ref):
    o_ref[...] = a_ref[...] + b_ref[...]


def _add2(a, b):
    m, d = a.shape
    return pl.pallas_call(
        _add2_body,
        grid=(m // _BM,),
        in_specs=[pl.BlockSpec((_BM, d), lambda i: (i, 0)),
                  pl.BlockSpec((_BM, d), lambda i: (i, 0))],
        out_specs=pl.BlockSpec((_BM, d), lambda i: (i, 0)),
        out_shape=jax.ShapeDtypeStruct((m, d), jnp.float32),
    )(a, b)


def _colsum_body(h_ref, o_ref):
    o_ref[...] = jnp.sum(h_ref[...], axis=0)


def _colsum(h):
    nw, m = h.shape
    return pl.pallas_call(
        _colsum_body,
        grid=(m // _BM,),
        in_specs=[pl.BlockSpec((nw, _BM), lambda i: (0, i))],
        out_specs=pl.BlockSpec((_BM,), lambda i: (i,)),
        out_shape=jax.ShapeDtypeStruct((m,), jnp.float32),
    )(h)


def _mm_body(a_ref, w_ref, scale_ref, bias_ref, attn_ref, out_ref, e_ref,
             *, has_scale, has_bias, has_attn):
    a = a_ref[...]
    if has_scale:
        s = lax.rsqrt(jnp.clip(jnp.sum(scale_ref[...], axis=0), 1.0, None))
        a = a * s[:, None]
    acc = jnp.dot(a, w_ref[...], preferred_element_type=jnp.float32)
    if has_bias:
        acc = acc + bias_ref[...][None, :]
    out_ref[...] = acc
    if has_attn:
        e_ref[...] = jnp.sum(acc * attn_ref[...][None, :], axis=1)


def _mm(a, w, *, scale=None, bias=None, attn=None):
    m, k = a.shape
    d = w.shape[1]
    has_scale = scale is not None
    has_bias = bias is not None
    has_attn = attn is not None
    grid = (m // _BM,)
    in_specs = [
        pl.BlockSpec((_BM, k), lambda i: (i, 0)),
        pl.BlockSpec((k, d), lambda i: (0, 0)),
        pl.BlockSpec((_NW, _BM), lambda i: (0, i)) if has_scale else pl.BlockSpec((128,), lambda i: (0,)),
        pl.BlockSpec((d,), lambda i: (0,)) if has_bias else pl.BlockSpec((128,), lambda i: (0,)),
        pl.BlockSpec((d,), lambda i: (0,)) if has_attn else pl.BlockSpec((128,), lambda i: (0,)),
    ]
    out_shapes = [jax.ShapeDtypeStruct((m, d), jnp.float32),
                  jax.ShapeDtypeStruct((m,), jnp.float32)]
    out_specs = [pl.BlockSpec((_BM, d), lambda i: (i, 0)),
                 pl.BlockSpec((_BM,), lambda i: (i,))]
    dummy = jnp.zeros((128,), jnp.float32)
    out, e = pl.pallas_call(
        functools.partial(_mm_body, has_scale=has_scale, has_bias=has_bias,
                          has_attn=has_attn),
        grid=grid,
        in_specs=in_specs,
        out_specs=out_specs,
        out_shape=out_shapes,
    )(a, w,
      scale if has_scale else dummy,
      bias if has_bias else dummy,
      attn if has_attn else dummy)
    if has_attn:
        return out, e
    return out


def _scale_bias_body(z_ref, s_ref, b_ref, o_ref):
    s = lax.rsqrt(jnp.clip(jnp.sum(s_ref[...], axis=0), 1.0, None))
    o_ref[...] = z_ref[...] * s[:, None] + b_ref[...][None, :]


def _scale_bias(z, s, b):
    m, d = z.shape
    return pl.pallas_call(
        _scale_bias_body,
        grid=(m // _BM,),
        in_specs=[pl.BlockSpec((_BM, d), lambda i: (i, 0)),
                  pl.BlockSpec((_NW, _BM), lambda i: (0, i)),
                  pl.BlockSpec((d,), lambda i: (0,))],
        out_specs=pl.BlockSpec((_BM, d), lambda i: (i, 0)),
        out_shape=jax.ShapeDtypeStruct((m, d), jnp.float32),
    )(z, s, b)


def _edge_softmax(e, dst, num_dst):
    m = jax.ops.segment_max(e, dst, num_segments=num_dst)
    m = jnp.where(jnp.isfinite(m), m, 0.0)
    ex = jnp.exp(e - m[dst])
    s = jax.ops.segment_sum(ex, dst, num_segments=num_dst)
    return ex / s[dst]


def kernel(feat_N, feat_Q, edge_nn, edge_np, edge_pq, W_gcn, b_gcn, W_src1,
           W_dst1, attn_l1, attn_r1, bias1, W_src2, W_dst2, attn_l2, attn_r2,
           bias2):
    n_n = feat_N.shape[0]
    mp = ((n_n + _BM - 1) // _BM) * _BM
    pad = mp - n_n
    featN_p = jnp.pad(feat_N, ((0, pad), (0, 0)))
    featQ_p = jnp.pad(feat_Q, ((0, pad), (0, 0)))

    def pad_edges(e2):
        s, d = e2[0], e2[1]
        epad = (-s.shape[0]) % (_NW * 128)
        sp = jnp.pad(s, (0, epad), constant_values=n_n)
        dp = jnp.pad(d, (0, epad), constant_values=mp - 1)
        return sp, dp, jnp.stack([sp, sp + mp])

    def split2(x):
        dh = x.shape[1] // 2
        return jnp.concatenate([x[:, :dh], x[:, dh:]], axis=0)

    def join2(x):
        mp_ = x.shape[0] // 2
        return jnp.concatenate([x[:mp_], x[mp_:]], axis=1)

    src_p, dst_p, src2 = pad_edges(edge_nn)
    hs_, hd_ = _sc_degree_hists(src_p, dst_p, mp)
    y = _mm(featN_p, W_gcn, scale=hs_)
    z = join2(_sc_gather_segsum(split2(y), src2, dst_p))
    h = _scale_bias(z, hd_, b_gcn)

    s1p, d1p, s1_2 = pad_edges(edge_np)
    hp = _sc_segmax_relu(h, s1p, d1p)

    fs, el = _mm(h, W_src1, attn=attn_l1)
    _fd, er = _mm(hp, W_dst1, attn=attn_r1)
    ex1, sh1 = _sc_edge_softmax_num(el, er, s1p, d1p)
    hp2 = join2(_sc_gather_segsum(split2(fs), s1_2, d1p, ex=ex1,
                                  s_sum=_colsum(sh1), bias=bias1))

    s2p, d2p, s2_2 = pad_edges(edge_pq)
    fs2, el2 = _mm(hp2, W_src2, attn=attn_l2)
    _fd2, er2 = _mm(featQ_p, W_dst2, attn=attn_r2)
    ex2, sh2 = _sc_edge_softmax_num(el2, er2, s2p, d2p)
    outf, a2p = _sc_gather_segsum(fs2, jnp.stack([s2p, s2p]), d2p, ex=ex2,
                                  s_sum=_colsum(sh2), bias=bias2, emit_a=True,
                                  edge_split=True)
    out = _add2(outf[:mp], outf[mp:])
    return out[:n_n], a2p[:edge_pq.shape[1]]


# speedup vs baseline: 5.1530x; 1.0254x over previous
import functools

import jax
import jax.numpy as jnp
from jax import lax
from jax.experimental import pallas as pl
from jax.experimental.pallas import tpu as pltpu
from jax.experimental.pallas import tpu_sc as plsc

_BM = 1024
_NC, _NS, _L = 2, 16, 16
_NW = _NC * _NS

_SC_PARAMS = pltpu.CompilerParams(needs_layout_passes=False)


@functools.cache
def _sc_mesh():
    return plsc.VectorSubcoreMesh(core_axis_name="c", subcore_axis_name="s",
                                  num_cores=_NC, num_subcores=_NS)


def _sc_degree_hists(src, dst, mp):
    e_pw = src.shape[0] // _NW

    @functools.partial(
        pl.kernel,
        out_type=[jax.ShapeDtypeStruct((_NW, mp), jnp.float32),
                  jax.ShapeDtypeStruct((_NW, mp), jnp.float32)],
        mesh=_sc_mesh(),
        compiler_params=_SC_PARAMS,
        scratch_types=[pltpu.VMEM((e_pw,), jnp.int32),
                       pltpu.VMEM((e_pw,), jnp.int32),
                       pltpu.VMEM((mp,), jnp.float32),
                       pltpu.VMEM((mp,), jnp.float32)],
    )
    def deg_kernel(src_hbm, dst_hbm, osrc_hbm, odst_hbm, sbuf, dbuf, hs, hd):
        c = lax.axis_index("c")
        s = lax.axis_index("s")
        w = s * _NC + c

        def zero_body(i, _):
            z = jnp.zeros((_L,), jnp.float32)
            hs[pl.ds(i * _L, _L)] = z
            hd[pl.ds(i * _L, _L)] = z
            return 0

        lax.fori_loop(0, mp // _L, zero_body, 0)
        pltpu.sync_copy(src_hbm.at[pl.ds(w * e_pw, e_pw)], sbuf)
        pltpu.sync_copy(dst_hbm.at[pl.ds(w * e_pw, e_pw)], dbuf)
        ones = jnp.full((_L,), 1.0, jnp.float32)

        def body(i, _):
            sv = sbuf[pl.ds(i * _L, _L)]
            dv = dbuf[pl.ds(i * _L, _L)]
            plsc.addupdate_scatter(hs, [sv], ones)
            plsc.addupdate_scatter(hd, [dv], ones)
            return 0

        lax.fori_loop(0, e_pw // _L, body, 0)
        pltpu.sync_copy(hs, osrc_hbm.at[w])
        pltpu.sync_copy(hd, odst_hbm.at[w])

    return deg_kernel(src, dst)


def _sc_gather_segsum(tab, src2, dst, ex=None, s_sum=None, bias=None,
                      emit_a=False, edge_split=False):
    if edge_split:
        mp, dh = tab.shape
        two_mp = 2 * mp
    else:
        two_mp, dh = tab.shape
        mp = two_mp // 2
    ep = dst.shape[0]
    t_pw = ep // _NW if edge_split else ep // _NS
    nch = t_pw // 128
    rpt = mp // _NS
    weighted = ex is not None
    has_bias = bias is not None

    ch = 80 if weighted else 128
    nch = t_pw // ch
    out_types = [jax.ShapeDtypeStruct((two_mp, dh), jnp.float32)]
    if emit_a:
        out_types.append(jax.ShapeDtypeStruct((ep,), jnp.float32))
    scratch = [pltpu.VMEM((ch,), jnp.int32),
               pltpu.VMEM((ch,), jnp.int32),
               pltpu.VMEM((ch,), jnp.int32),
               pltpu.VMEM((ch,), jnp.int32),
               pltpu.VMEM((ch, dh), jnp.float32),
               pltpu.VMEM((ch, dh), jnp.float32),
               pltpu.VMEM((16, dh), jnp.float32),
               pltpu.SemaphoreType.DMA,
               pltpu.SemaphoreType.DMA,
               pltpu.SemaphoreType.DMA,
               pltpu.SemaphoreType.DMA]
    if weighted:
        scratch += [pltpu.VMEM((ch,), jnp.float32),
                    pltpu.VMEM((ch,), jnp.float32),
                    pltpu.VMEM((t_pw,), jnp.float32),
                    pltpu.VMEM((mp,), jnp.float32)]
    if has_bias:
        scratch.append(pltpu.VMEM((dh,), jnp.float32))
    scratch.append(pltpu.VMEM_SHARED((mp, dh), jnp.float32))

    def body(*args):
        if weighted:
            if has_bias:
                (tab_h, src2_h, dst_h, ex_h, ssum_h, bias_h), rest = args[:6], args[6:]
            else:
                (tab_h, src2_h, dst_h, ex_h, ssum_h), rest = args[:5], args[5:]
                bias_h = None
        else:
            if has_bias:
                (tab_h, src2_h, dst_h, bias_h), rest = args[:4], args[4:]
            else:
                (tab_h, src2_h, dst_h), rest = args[:3], args[3:]
                bias_h = None
            ex_h = ssum_h = None
        if emit_a:
            out_h, a_h = rest[0], rest[1]
            rest = rest[2:]
        else:
            out_h = rest[0]
            a_h = None
            rest = rest[1:]
        (sidx0, sidx1, didx0, didx1, rows0, rows1, zbuf,
         smi0, smi1, smr0, smr1) = rest[:11]
        rest = rest[11:]
        eb0 = eb1 = abuf = sv = biasv = None
        if weighted:
            eb0, eb1, abuf, sv = rest[:4]
            rest = rest[4:]
        if has_bias:
            biasv = rest[0]
            rest = rest[1:]
        acc = rest[0]

        c = lax.axis_index("c")
        s = lax.axis_index("s")

        zero16 = jnp.zeros((_L,), jnp.float32)
        if has_bias and edge_split:
            pltpu.sync_copy(bias_h, biasv)
            bvals = [jnp.where(c == 0, biasv[pl.ds(j * _L, _L)], zero16)
                     for j in range(dh // _L)]
        elif has_bias:
            pltpu.sync_copy(bias_h.at[pl.ds(c * dh, dh)], biasv)
            bvals = [biasv[pl.ds(j * _L, _L)] for j in range(dh // _L)]
        else:
            bvals = [zero16] * (dh // _L)

        def zrow(i, _):
            for j in range(dh // _L):
                zbuf[i, pl.ds(j * _L, _L)] = bvals[j]
            return 0

        lax.fori_loop(0, 16, zrow, 0)
        for k in range(rpt // 16):
            pltpu.sync_copy(zbuf, acc.at[pl.ds(s * rpt + k * 16, 16)])
        plsc.subcore_barrier()

        if weighted:
            pltpu.sync_copy(ssum_h, sv)

        base = c * (ep // 2) + s * t_pw if edge_split else s * t_pw

        def idx_start(g, sb, db, eb, sm):
            off = base + g * ch
            pltpu.async_copy(src2_h.at[pl.ds(c * ep + off, ch)], sb, sm)
            pltpu.async_copy(dst_h.at[pl.ds(off, ch)], db, sm)
            if weighted:
                pltpu.async_copy(ex_h.at[pl.ds(off, ch)], eb, sm)

        def idx_wait(g, sb, db, eb, sm):
            off = base + g * ch
            pltpu.make_async_copy(src2_h.at[pl.ds(c * ep + off, ch)], sb, sm).wait()
            pltpu.make_async_copy(dst_h.at[pl.ds(off, ch)], db, sm).wait()
            if weighted:
                pltpu.make_async_copy(ex_h.at[pl.ds(off, ch)], eb, sm).wait()

        def gather_start(sb, rows, sm):
            pltpu.async_copy(tab_h.at[sb], rows, sm)

        def gather_wait(sb, rows, sm):
            pltpu.make_async_copy(tab_h.at[sb], rows, sm).wait()

        def process(g, db, eb, rows):
            if weighted:
                for j in range(ch // _L):
                    dv = db[pl.ds(j * _L, _L)]
                    aw = eb[pl.ds(j * _L, _L)] / plsc.load_gather(sv, [dv])
                    abuf[pl.ds(g * ch + j * _L, _L)] = aw

                def escale(e, _):
                    aw = plsc.load_gather(
                        abuf, [jnp.full((_L,), 0, jnp.int32) + (g * ch + e)])
                    scaled = [rows[e, pl.ds(j * _L, _L)] * aw
                              for j in range(dh // _L)]
                    for j in range(dh // _L):
                        rows[e, pl.ds(j * _L, _L)] = scaled[j]
                    return 0

                lax.fori_loop(0, ch, escale, 0)
            pltpu.sync_copy(rows, acc.at[db], add=True)

        idx_start(0, sidx0, didx0, eb0, smi0)
        idx_wait(0, sidx0, didx0, eb0, smi0)
        gather_start(sidx0, rows0, smr0)
        idx_start(1, sidx1, didx1, eb1, smi1)

        def pair(gp, _):
            g0 = 2 * gp
            gather_wait(sidx0, rows0, smr0)
            idx_wait(g0 + 1, sidx1, didx1, eb1, smi1)
            gather_start(sidx1, rows1, smr1)
            process(g0, didx0, eb0, rows0)

            @pl.when(g0 + 2 < nch)
            def _():
                idx_start(g0 + 2, sidx0, didx0, eb0, smi0)

            gather_wait(sidx1, rows1, smr1)

            @pl.when(g0 + 2 < nch)
            def _():
                idx_wait(g0 + 2, sidx0, didx0, eb0, smi0)
                gather_start(sidx0, rows0, smr0)

            process(g0 + 1, didx1, eb1, rows1)

            @pl.when(g0 + 3 < nch)
            def _():
                idx_start(g0 + 3, sidx1, didx1, eb1, smi1)

            return 0

        lax.fori_loop(0, nch // 2, pair, 0)

        if weighted and emit_a and edge_split:
            pltpu.sync_copy(abuf, a_h.at[pl.ds(base, t_pw)])
        elif weighted and emit_a:
            @pl.when(c == 0)
            def _():
                pltpu.sync_copy(abuf, a_h.at[pl.ds(base, t_pw)])
        plsc.subcore_barrier()
        for k in range(rpt // 128):
            r0 = s * rpt + k * 128
            pltpu.sync_copy(acc.at[pl.ds(r0, 128)],
                            out_h.at[pl.ds(c * mp + r0, 128)])

    fn = pl.kernel(body, out_type=out_types, mesh=_sc_mesh(),
                   compiler_params=_SC_PARAMS, scratch_types=scratch)
    ins = [tab, src2.reshape(-1), dst]
    if weighted:
        ins += [ex, s_sum]
    if has_bias:
        ins.append(bias)
    res = fn(*ins)
    return res if emit_a else res[0]


def _sc_edge_softmax_num(el, er, src, dst):
    mp = el.shape[0]
    ep = src.shape[0]
    e_pw = ep // _NW

    def body(el_h, er_h, src_h, dst_h, ex_h, hist_h, elv, erv, sbuf, dbuf,
             exbuf, hist):
        c = lax.axis_index("c")
        s = lax.axis_index("s")
        w = s * _NC + c

        def zero_body(i, _):
            hist[pl.ds(i * _L, _L)] = jnp.zeros((_L,), jnp.float32)
            return 0

        lax.fori_loop(0, mp // _L, zero_body, 0)
        pltpu.sync_copy(el_h, elv)
        pltpu.sync_copy(er_h, erv)
        pltpu.sync_copy(src_h.at[pl.ds(w * e_pw, e_pw)], sbuf)
        pltpu.sync_copy(dst_h.at[pl.ds(w * e_pw, e_pw)], dbuf)

        def body_i(i, _):
            sv = sbuf[pl.ds(i * _L, _L)]
            dv = dbuf[pl.ds(i * _L, _L)]
            x = plsc.load_gather(elv, [sv]) + plsc.load_gather(erv, [dv])
            x = jnp.maximum(x, 0.2 * x)
            exv = jnp.exp(x)
            exbuf[pl.ds(i * _L, _L)] = exv
            plsc.addupdate_scatter(hist, [dv], exv)
            return 0

        lax.fori_loop(0, e_pw // _L, body_i, 0)
        pltpu.sync_copy(exbuf, ex_h.at[pl.ds(w * e_pw, e_pw)])
        pltpu.sync_copy(hist, hist_h.at[w])

    fn = pl.kernel(
        body,
        out_type=[jax.ShapeDtypeStruct((ep,), jnp.float32),
                  jax.ShapeDtypeStruct((_NW, mp), jnp.float32)],
        mesh=_sc_mesh(),
        compiler_params=_SC_PARAMS,
        scratch_types=[pltpu.VMEM((mp,), jnp.float32),
                       pltpu.VMEM((mp,), jnp.float32),
                       pltpu.VMEM((e_pw,), jnp.int32),
                       pltpu.VMEM((e_pw,), jnp.int32),
                       pltpu.VMEM((e_pw,), jnp.float32),
                       pltpu.VMEM((mp,), jnp.float32)])
    return fn(el, er, src, dst)


def _sc_segmax_relu(tab, src, dst):
    mp, d = tab.shape
    ep = src.shape[0]
    rpw = mp // _NW
    ce = 2048
    nst = ep // ce
    ndj = d // _L

    def body(tab_h, src_h, dst_h, out_h, sbuf0, dbuf0, sbuf1, dbuf1, msrc,
             mdl, grows0, grows1, acc, semG0, semG1, semA, semB):
        c = lax.axis_index("c")
        s = lax.axis_index("s")
        w = s * _NC + c
        lo = w * rpw

        def zrow(i, _):
            for j in range(ndj):
                acc[i, pl.ds(j * _L, _L)] = jnp.zeros((_L,), jnp.float32)
            return 0

        lax.fori_loop(0, rpw, zrow, 0)

        def zidx(i, _):
            msrc[pl.ds(i * _L, _L)] = jnp.zeros((_L,), jnp.int32)
            mdl[pl.ds(i * _L, _L)] = jnp.zeros((_L,), jnp.int32)
            return 0

        lax.fori_loop(0, ce // _L, zidx, 0)

        cols = [lax.iota(jnp.int32, _L) + j * _L for j in range(ndj)]

        def stage_start(st, sb, db, sm):
            soff = st * ce
            pltpu.async_copy(src_h.at[pl.ds(soff, ce)], sb, sm)
            pltpu.async_copy(dst_h.at[pl.ds(soff, ce)], db, sm)

        def stage(st, sbuf, dbuf, sm):
            soff = st * ce
            pltpu.make_async_copy(src_h.at[pl.ds(soff, ce)], sbuf, sm).wait()
            pltpu.make_async_copy(dst_h.at[pl.ds(soff, ce)], dbuf, sm).wait()

            def filt(j, cnt):
                sv = sbuf[pl.ds(j * _L, _L)]
                dl = dbuf[pl.ds(j * _L, _L)] - lo
                mask = (dl >= 0) & (dl < rpw)
                pos = cnt + plsc.cumsum(jnp.where(mask, 1, 0)) - 1
                plsc.store_scatter(msrc, [pos], sv, mask=mask)
                plsc.store_scatter(mdl, [pos], dl, mask=mask)
                return cnt + plsc.all_reduce_population_count(mask)

            cntv = lax.fori_loop(0, ce // _L, filt,
                                 jnp.zeros((_L,), jnp.int32))
            cnt = jnp.max(cntv)

            gr = 32
            ng = (cnt + gr - 1) // gr

            def g_start(gi, grows, sm):
                pltpu.async_copy(tab_h.at[msrc.at[pl.ds(gi * gr, gr)]],
                                 grows, sm)

            def g_wait(gi, grows, sm):
                pltpu.make_async_copy(tab_h.at[msrc.at[pl.ds(gi * gr, gr)]],
                                      grows, sm).wait()

            def g_upd(gi, grows):
                gb = gi * gr
                nloc = jnp.minimum(gr, cnt - gb)

                def upd(e, _):
                    dlw = plsc.load_gather(
                        mdl, [jnp.full((_L,), 0, jnp.int32) + (gb + e)])
                    news = [jnp.maximum(plsc.load_gather(acc, [dlw, cols[j]]),
                                        grows[e, pl.ds(j * _L, _L)])
                            for j in range(ndj)]
                    for j in range(ndj):
                        plsc.store_scatter(acc, [dlw, cols[j]], news[j])
                    return 0

                lax.fori_loop(0, nloc, upd, 0)

            @pl.when(ng > 0)
            def _():
                g_start(0, grows0, semG0)

            def gpair(q, _):
                g0 = 2 * q

                @pl.when(g0 + 1 < ng)
                def _():
                    g_start(g0 + 1, grows1, semG1)

                g_wait(g0, grows0, semG0)
                g_upd(g0, grows0)

                @pl.when(g0 + 2 < ng)
                def _():
                    g_start(g0 + 2, grows0, semG0)

                @pl.when(g0 + 1 < ng)
                def _():
                    g_wait(g0 + 1, grows1, semG1)
                    g_upd(g0 + 1, grows1)

                return 0

            lax.fori_loop(0, (ng + 1) // 2, gpair, 0)

        stage_start(0, sbuf0, dbuf0, semA)

        def spair(sp, _):
            st0 = 2 * sp
            stage_start(st0 + 1, sbuf1, dbuf1, semB)
            stage(st0, sbuf0, dbuf0, semA)

            @pl.when(sp + 1 < nst // 2)
            def _():
                stage_start(st0 + 2, sbuf0, dbuf0, semA)

            stage(st0 + 1, sbuf1, dbuf1, semB)
            return 0

        lax.fori_loop(0, nst // 2, spair, 0)
        pltpu.sync_copy(acc, out_h.at[pl.ds(lo, rpw)])

    fn = pl.kernel(
        body,
        out_type=jax.ShapeDtypeStruct((mp, d), jnp.float32),
        mesh=_sc_mesh(),
        compiler_params=_SC_PARAMS,
        scratch_types=[pltpu.VMEM((ce,), jnp.int32),
                       pltpu.VMEM((ce,), jnp.int32),
                       pltpu.VMEM((ce,), jnp.int32),
                       pltpu.VMEM((ce,), jnp.int32),
                       pltpu.VMEM((ce,), jnp.int32),
                       pltpu.VMEM((ce,), jnp.int32),
                       pltpu.VMEM((32, d), jnp.float32),
                       pltpu.VMEM((32, d), jnp.float32),
                       pltpu.VMEM((rpw, d), jnp.float32),
                       pltpu.SemaphoreType.DMA,
                       pltpu.SemaphoreType.DMA,
                       pltpu.SemaphoreType.DMA,
                       pltpu.SemaphoreType.DMA])
    return fn(tab, src, dst)


def _add2_body(a_ref, b_ref, o_ref):
    o_ref[...] = a_ref[...] + b_ref[...]


def _add2(a, b):
    m, d = a.shape
    return pl.pallas_call(
        _add2_body,
        grid=(m // _BM,),
        in_specs=[pl.BlockSpec((_BM, d), lambda i: (i, 0)),
                  pl.BlockSpec((_BM, d), lambda i: (i, 0))],
        out_specs=pl.BlockSpec((_BM, d), lambda i: (i, 0)),
        out_shape=jax.ShapeDtypeStruct((m, d), jnp.float32),
    )(a, b)


def _colsum_body(h_ref, o_ref):
    o_ref[...] = jnp.sum(h_ref[...], axis=0)


def _colsum(h):
    nw, m = h.shape
    return pl.pallas_call(
        _colsum_body,
        grid=(m // _BM,),
        in_specs=[pl.BlockSpec((nw, _BM), lambda i: (0, i))],
        out_specs=pl.BlockSpec((_BM,), lambda i: (i,)),
        out_shape=jax.ShapeDtypeStruct((m,), jnp.float32),
    )(h)


def _mm_body(a_ref, w_ref, scale_ref, bias_ref, attn_ref, out_ref, e_ref,
             *, has_scale, has_bias, has_attn):
    a = a_ref[...]
    if has_scale:
        s = lax.rsqrt(jnp.clip(jnp.sum(scale_ref[...], axis=0), 1.0, None))
        a = a * s[:, None]
    acc = jnp.dot(a, w_ref[...], preferred_element_type=jnp.float32)
    if has_bias:
        acc = acc + bias_ref[...][None, :]
    out_ref[...] = acc
    if has_attn:
        e_ref[...] = jnp.sum(acc * attn_ref[...][None, :], axis=1)


def _mm(a, w, *, scale=None, bias=None, attn=None):
    m, k = a.shape
    d = w.shape[1]
    has_scale = scale is not None
    has_bias = bias is not None
    has_attn = attn is not None
    grid = (m // _BM,)
    in_specs = [
        pl.BlockSpec((_BM, k), lambda i: (i, 0)),
        pl.BlockSpec((k, d), lambda i: (0, 0)),
        pl.BlockSpec((_NW, _BM), lambda i: (0, i)) if has_scale else pl.BlockSpec((128,), lambda i: (0,)),
        pl.BlockSpec((d,), lambda i: (0,)) if has_bias else pl.BlockSpec((128,), lambda i: (0,)),
        pl.BlockSpec((d,), lambda i: (0,)) if has_attn else pl.BlockSpec((128,), lambda i: (0,)),
    ]
    out_shapes = [jax.ShapeDtypeStruct((m, d), jnp.float32),
                  jax.ShapeDtypeStruct((m,), jnp.float32)]
    out_specs = [pl.BlockSpec((_BM, d), lambda i: (i, 0)),
                 pl.BlockSpec((_BM,), lambda i: (i,))]
    dummy = jnp.zeros((128,), jnp.float32)
    out, e = pl.pallas_call(
        functools.partial(_mm_body, has_scale=has_scale, has_bias=has_bias,
                          has_attn=has_attn),
        grid=grid,
        in_specs=in_specs,
        out_specs=out_specs,
        out_shape=out_shapes,
    )(a, w,
      scale if has_scale else dummy,
      bias if has_bias else dummy,
      attn if has_attn else dummy)
    if has_attn:
        return out, e
    return out


def _scale_bias_body(z_ref, s_ref, b_ref, o_ref):
    s = lax.rsqrt(jnp.clip(jnp.sum(s_ref[...], axis=0), 1.0, None))
    o_ref[...] = z_ref[...] * s[:, None] + b_ref[...][None, :]


def _scale_bias(z, s, b):
    m, d = z.shape
    return pl.pallas_call(
        _scale_bias_body,
        grid=(m // _BM,),
        in_specs=[pl.BlockSpec((_BM, d), lambda i: (i, 0)),
                  pl.BlockSpec((_NW, _BM), lambda i: (0, i)),
                  pl.BlockSpec((d,), lambda i: (0,))],
        out_specs=pl.BlockSpec((_BM, d), lambda i: (i, 0)),
        out_shape=jax.ShapeDtypeStruct((m, d), jnp.float32),
    )(z, s, b)


def _edge_softmax(e, dst, num_dst):
    m = jax.ops.segment_max(e, dst, num_segments=num_dst)
    m = jnp.where(jnp.isfinite(m), m, 0.0)
    ex = jnp.exp(e - m[dst])
    s = jax.ops.segment_sum(ex, dst, num_segments=num_dst)
    return ex / s[dst]


def kernel(feat_N, feat_Q, edge_nn, edge_np, edge_pq, W_gcn, b_gcn, W_src1,
           W_dst1, attn_l1, attn_r1, bias1, W_src2, W_dst2, attn_l2, attn_r2,
           bias2):
    n_n = feat_N.shape[0]
    mp = ((n_n + _BM - 1) // _BM) * _BM
    pad = mp - n_n
    featN_p = jnp.pad(feat_N, ((0, pad), (0, 0)))
    featQ_p = jnp.pad(feat_Q, ((0, pad), (0, 0)))

    def pad_edges(e2):
        s, d = e2[0], e2[1]
        epad = (-s.shape[0]) % (_NW * 128)
        sp = jnp.pad(s, (0, epad), constant_values=n_n)
        dp = jnp.pad(d, (0, epad), constant_values=mp - 1)
        return sp, dp, jnp.stack([sp, sp + mp])

    def split2(x):
        dh = x.shape[1] // 2
        return jnp.concatenate([x[:, :dh], x[:, dh:]], axis=0)

    def join2(x):
        mp_ = x.shape[0] // 2
        return jnp.concatenate([x[:mp_], x[mp_:]], axis=1)

    src_p, dst_p, src2 = pad_edges(edge_nn)
    hs_, hd_ = _sc_degree_hists(src_p, dst_p, mp)
    y = _mm(featN_p, W_gcn, scale=hs_)
    z = join2(_sc_gather_segsum(split2(y), src2, dst_p))
    h = _scale_bias(z, hd_, b_gcn)

    s1p, d1p, s1_2 = pad_edges(edge_np)
    hp = _sc_segmax_relu(h, s1p, d1p)

    fs, el = _mm(h, W_src1, attn=attn_l1)
    _fd, er = _mm(hp, W_dst1, attn=attn_r1)
    ex1, sh1 = _sc_edge_softmax_num(el, er, s1p, d1p)
    hp2 = join2(_sc_gather_segsum(split2(fs), s1_2, d1p, ex=ex1,
                                  s_sum=_colsum(sh1), bias=bias1))

    s2p, d2p, s2_2 = pad_edges(edge_pq)
    fs2, el2 = _mm(hp2, W_src2, attn=attn_l2)
    _fd2, er2 = _mm(featQ_p, W_dst2, attn=attn_r2)
    ex2, sh2 = _sc_edge_softmax_num(el2, er2, s2p, d2p)
    outf, a2p = _sc_gather_segsum(fs2, jnp.stack([s2p, s2p]), d2p, ex=ex2,
                                  s_sum=_colsum(sh2), bias=bias2, emit_a=True,
                                  edge_split=True)
    out = _add2(outf[:mp], outf[mp:])
    return out[:n_n], a2p[:edge_pq.shape[1]]
